# baseline (speedup 1.0000x reference)
import functools

import jax
import jax.numpy as jnp
from jax import lax
from jax.experimental import pallas as pl
from jax.experimental.pallas import tpu as pltpu
from jax.experimental.pallas import tpu_sc as plsc

N_NODES = 10000
N_EDGES = 320000
F_IN = 128
F_HID = 64
N_CLS = 3
N_GRAPHS = 64

NC = 2
NS = 16
NW = NC * NS
N_PAD = 10240
RPS = N_PAD // NS
CHUNK = 128
CPW = 80
EPW = CPW * CHUNK
E_PAD = NW * EPW
NBUF = 4
ROW_BLK = 1024

_mesh = functools.partial(
    plsc.VectorSubcoreMesh, core_axis_name="c", subcore_axis_name="s"
)
_sc_params = pltpu.CompilerParams(use_tc_tiling_on_sc=False)


def _sc_degree(dst3, onz):
    @functools.partial(
        pl.kernel,
        mesh=_mesh(),
        compiler_params=_sc_params,
        out_type=jax.ShapeDtypeStruct((NC * N_PAD,), jnp.float32),
        scratch_types=[
            pltpu.VMEM((CPW, CHUNK), jnp.int32),
            pltpu.VMEM((CHUNK,), jnp.float32),
            pltpu.VMEM((CHUNK,), jnp.float32),
            pltpu.VMEM_SHARED((N_PAD,), jnp.float32),
            pltpu.SemaphoreType.DMA,
        ],
    )
    def k(dst_hbm, onz_hbm, out_hbm, idx_a, ones_v, stage_v, acc_sh, sem):
        c = lax.axis_index("c")
        s = lax.axis_index("s")
        wid = s * NC + c
        pltpu.sync_copy(onz_hbm.at[pl.ds(0, CHUNK)], ones_v)
        pltpu.sync_copy(onz_hbm.at[pl.ds(CHUNK, CHUNK)], stage_v)

        def zbody(j, carry):
            pltpu.sync_copy(
                stage_v, acc_sh.at[pl.ds(s * RPS + j * CHUNK, CHUNK)]
            )
            return carry

        lax.fori_loop(0, RPS // CHUNK, zbody, 0)
        pltpu.sync_copy(dst_hbm.at[wid], idx_a)
        plsc.subcore_barrier()

        def body(o, carry):
            for j in range(8):
                pltpu.async_copy(
                    ones_v, acc_sh.at[idx_a.at[o * 8 + j]], sem, add=True
                )
            for j in range(8):
                pltpu.make_async_copy(
                    ones_v, acc_sh.at[idx_a.at[0]], sem
                ).wait()
            return carry

        lax.fori_loop(0, CPW // 8, body, 0)
        plsc.subcore_barrier()

        def obody(j, carry):
            off = s * RPS + j * CHUNK
            pltpu.sync_copy(acc_sh.at[pl.ds(off, CHUNK)], stage_v)
            pltpu.sync_copy(stage_v, out_hbm.at[pl.ds(c * N_PAD + off, CHUNK)])
            return carry

        lax.fori_loop(0, RPS // CHUNK, obody, 0)

    return k(dst3, onz)


def _sc_agg(g, src3, dst3, zeros_rows):
    @functools.partial(
        pl.kernel,
        mesh=_mesh(),
        compiler_params=_sc_params,
        out_type=jax.ShapeDtypeStruct((NC * N_PAD, F_HID), jnp.float32),
        scratch_types=[
            pltpu.VMEM((CPW, CHUNK), jnp.int32),
            pltpu.VMEM((CPW, CHUNK), jnp.int32),
            [pltpu.VMEM((CHUNK, F_HID), jnp.float32)] * NBUF,
            [pltpu.SemaphoreType.DMA] * NBUF,
            [pltpu.SemaphoreType.DMA] * NBUF,
            pltpu.VMEM_SHARED((N_PAD, F_HID), jnp.float32),
        ],
    )
    def k(g_hbm, src_hbm, dst_hbm, z_hbm, out_hbm,
          src_a, dst_a, rows, gsem, ssem, acc_sh):
        c = lax.axis_index("c")
        s = lax.axis_index("s")
        wid = s * NC + c

        pltpu.sync_copy(z_hbm, rows[0])

        def zbody(j, carry):
            pltpu.sync_copy(
                rows[0], acc_sh.at[pl.ds(s * RPS + j * CHUNK, CHUNK)]
            )
            return carry

        lax.fori_loop(0, RPS // CHUNK, zbody, 0)

        pltpu.sync_copy(src_hbm.at[wid], src_a)
        pltpu.sync_copy(dst_hbm.at[wid], dst_a)
        plsc.subcore_barrier()

        for b in range(NBUF):
            pltpu.async_copy(g_hbm.at[src_a.at[b]], rows[b], gsem[b])

        def outer(o, carry):
            for b in range(NBUF):
                i = o * NBUF + b
                pltpu.make_async_copy(
                    g_hbm.at[src_a.at[0]], rows[b], gsem[b]
                ).wait()
                pltpu.async_copy(
                    rows[b], acc_sh.at[dst_a.at[i]], ssem[b], add=True
                )
                pltpu.make_async_copy(
                    rows[b], acc_sh.at[dst_a.at[i]], ssem[b]
                ).wait()

                @pl.when(i + NBUF < CPW)
                def _():
                    pltpu.async_copy(
                        g_hbm.at[src_a.at[i + NBUF]], rows[b], gsem[b]
                    )

            return carry

        lax.fori_loop(0, CPW // NBUF, outer, 0)
        plsc.subcore_barrier()

        def obody(j, carry):
            off = s * RPS + j * CHUNK
            pltpu.sync_copy(acc_sh.at[pl.ds(off, CHUNK)], rows[0])
            pltpu.sync_copy(rows[0], out_hbm.at[pl.ds(c * N_PAD + off, CHUNK)])
            return carry

        lax.fori_loop(0, RPS // CHUNK, obody, 0)

    return k(g, src3, dst3, zeros_rows)


def _tc1_body(degp_ref, x_ref, w1_ref, dis_ref, g1_ref):
    deg = degp_ref[:, 0:1] + degp_ref[:, 1:2] + 1.0
    dis = lax.rsqrt(deg)
    dis_ref[...] = dis
    z = jnp.dot(x_ref[...], w1_ref[...], preferred_element_type=jnp.float32)
    g1_ref[...] = z * dis


def _tc1(degp, x, w1):
    grid = N_PAD // ROW_BLK
    return pl.pallas_call(
        _tc1_body,
        grid=(grid,),
        in_specs=[
            pl.BlockSpec((ROW_BLK, 2), lambda i: (i, 0)),
            pl.BlockSpec((ROW_BLK, F_IN), lambda i: (i, 0)),
            pl.BlockSpec((F_IN, F_HID), lambda i: (0, 0)),
        ],
        out_specs=[
            pl.BlockSpec((ROW_BLK, 1), lambda i: (i, 0)),
            pl.BlockSpec((ROW_BLK, F_HID), lambda i: (i, 0)),
        ],
        out_shape=[
            jax.ShapeDtypeStruct((N_PAD, 1), jnp.float32),
            jax.ShapeDtypeStruct((N_PAD, F_HID), jnp.float32),
        ],
    )(degp, x, w1)


def _tc2_body(a0_ref, a1_ref, g1_ref, dis_ref, b1_ref, w2_ref, g2_ref):
    h1 = jnp.maximum(
        (a0_ref[...] + a1_ref[...] + g1_ref[...]) * dis_ref[...] + b1_ref[...],
        0.0,
    )
    z2 = jnp.dot(h1, w2_ref[...], preferred_element_type=jnp.float32)
    g2_ref[...] = z2 * dis_ref[...]


def _tc2(a0, a1, g1, dis, b1, w2):
    grid = N_PAD // ROW_BLK
    rb = pl.BlockSpec((ROW_BLK, F_HID), lambda i: (i, 0))
    return pl.pallas_call(
        _tc2_body,
        grid=(grid,),
        in_specs=[
            rb,
            rb,
            rb,
            pl.BlockSpec((ROW_BLK, 1), lambda i: (i, 0)),
            pl.BlockSpec((1, F_HID), lambda i: (0, 0)),
            pl.BlockSpec((F_HID, F_HID), lambda i: (0, 0)),
        ],
        out_specs=rb,
        out_shape=jax.ShapeDtypeStruct((N_PAD, F_HID), jnp.float32),
    )(a0, a1, g1, dis, b1, w2)


def _tc3_body(
    a0_ref, a1_ref, g2_ref, dis_ref, b2_ref, batch_ref, wl_ref, bl_ref,
    out_ref, pooled_ref, cnt_ref,
):
    i = pl.program_id(0)

    @pl.when(i == 0)
    def _():
        pooled_ref[...] = jnp.zeros_like(pooled_ref)
        cnt_ref[...] = jnp.zeros_like(cnt_ref)

    h2 = jnp.maximum(
        (a0_ref[...] + a1_ref[...] + g2_ref[...]) * dis_ref[...] + b2_ref[...],
        0.0,
    )
    ids = batch_ref[...]
    onehot = (
        ids == lax.broadcasted_iota(jnp.int32, (1, N_GRAPHS), 1)
    ).astype(jnp.float32)
    dn = (((0,), (0,)), ((), ()))
    pooled_ref[...] += lax.dot_general(
        onehot, h2, dn, preferred_element_type=jnp.float32
    )
    cnt_ref[...] += lax.dot_general(
        onehot,
        jnp.ones((ROW_BLK, 1), jnp.float32),
        dn,
        preferred_element_type=jnp.float32,
    )

    @pl.when(i == pl.num_programs(0) - 1)
    def _():
        mean = pooled_ref[...] / jnp.maximum(cnt_ref[...], 1.0)
        out_ref[...] = (
            jnp.dot(mean, wl_ref[...], preferred_element_type=jnp.float32)
            + bl_ref[...]
        )


def _tc3(a0, a1, g2, dis, b2, batchp, wl, bl):
    grid = N_PAD // ROW_BLK
    rb = pl.BlockSpec((ROW_BLK, F_HID), lambda i: (i, 0))
    return pl.pallas_call(
        _tc3_body,
        grid=(grid,),
        in_specs=[
            rb,
            rb,
            rb,
            pl.BlockSpec((ROW_BLK, 1), lambda i: (i, 0)),
            pl.BlockSpec((1, F_HID), lambda i: (0, 0)),
            pl.BlockSpec((ROW_BLK, 1), lambda i: (i, 0)),
            pl.BlockSpec((F_HID, N_CLS), lambda i: (0, 0)),
            pl.BlockSpec((1, N_CLS), lambda i: (0, 0)),
        ],
        out_specs=pl.BlockSpec((N_GRAPHS, N_CLS), lambda i: (0, 0)),
        out_shape=jax.ShapeDtypeStruct((N_GRAPHS, N_CLS), jnp.float32),
        scratch_shapes=[
            pltpu.VMEM((N_GRAPHS, N_GRAPHS), jnp.float32),
            pltpu.VMEM((N_GRAPHS, 1), jnp.float32),
        ],
    )(a0, a1, g2, dis, b2, batchp, wl, bl)


def kernel(x, edge_index, batch, W1, b1, W2, b2, Wl, bl):
    pad_idx = N_PAD - 1
    src3 = jnp.pad(
        edge_index[0], (0, E_PAD - N_EDGES), constant_values=pad_idx
    ).reshape(NW, CPW, CHUNK)
    dst3 = jnp.pad(
        edge_index[1], (0, E_PAD - N_EDGES), constant_values=pad_idx
    ).reshape(NW, CPW, CHUNK)

    x_p = jnp.pad(x, ((0, N_PAD - N_NODES), (0, 0)))
    batch_p = jnp.pad(
        batch, (0, N_PAD - N_NODES), constant_values=N_GRAPHS
    ).reshape(N_PAD, 1)

    onz = jnp.concatenate(
        [jnp.ones((CHUNK,), jnp.float32), jnp.zeros((CHUNK,), jnp.float32)]
    )
    zeros_rows = jnp.zeros((CHUNK, F_HID), jnp.float32)

    deg_flat = _sc_degree(dst3, onz)
    degp = deg_flat.reshape(NC, N_PAD).T

    dis, g1 = _tc1(degp, x_p, W1)

    acc1 = _sc_agg(g1, src3, dst3, zeros_rows)
    g2 = _tc2(acc1[:N_PAD], acc1[N_PAD:], g1, dis, b1.reshape(1, F_HID), W2)

    acc2 = _sc_agg(g2, src3, dst3, zeros_rows)
    out = _tc3(
        acc2[:N_PAD],
        acc2[N_PAD:],
        g2,
        dis,
        b2.reshape(1, F_HID),
        batch_p,
        Wl,
        bl.reshape(1, N_CLS),
    )
    return out


# speedup vs baseline: 2.3103x; 2.3103x over previous
import functools

import jax
import jax.numpy as jnp
from jax import lax
from jax.experimental import pallas as pl
from jax.experimental.pallas import tpu as pltpu
from jax.experimental.pallas import tpu_sc as plsc

N_NODES = 10000
N_EDGES = 320000
F_IN = 128
F_HID = 64
HALF = F_HID // 2
N_CLS = 3
N_GRAPHS = 64

NC = 2
NS = 16
NW = NC * NS
N_PAD = 10240
RPS = N_PAD // NS
CHUNK = 128
CPT = 160
EPT = CPT * CHUNK
E_PAD = NS * EPT
NBUF = 4
ROW_BLK = 1024

_mesh = functools.partial(
    plsc.VectorSubcoreMesh, core_axis_name="c", subcore_axis_name="s"
)
_sc_params = pltpu.CompilerParams(use_tc_tiling_on_sc=False)


def _sc_degree(dst3, onz):
    @functools.partial(
        pl.kernel,
        mesh=_mesh(),
        compiler_params=_sc_params,
        out_type=jax.ShapeDtypeStruct((NC * N_PAD,), jnp.float32),
        scratch_types=[
            pltpu.VMEM((CPT // 2, CHUNK), jnp.int32),
            pltpu.VMEM((CHUNK,), jnp.float32),
            pltpu.VMEM((CHUNK,), jnp.float32),
            pltpu.VMEM_SHARED((N_PAD,), jnp.float32),
            pltpu.SemaphoreType.DMA,
        ],
    )
    def k(dst_hbm, onz_hbm, out_hbm, idx_a, ones_v, stage_v, acc_sh, sem):
        c = lax.axis_index("c")
        s = lax.axis_index("s")
        pltpu.sync_copy(onz_hbm.at[pl.ds(0, CHUNK)], ones_v)
        pltpu.sync_copy(onz_hbm.at[pl.ds(CHUNK, CHUNK)], stage_v)

        def zbody(j, carry):
            pltpu.sync_copy(
                stage_v, acc_sh.at[pl.ds(s * RPS + j * CHUNK, CHUNK)]
            )
            return carry

        lax.fori_loop(0, RPS // CHUNK, zbody, 0)
        pltpu.sync_copy(dst_hbm.at[s].at[pl.ds(c * (CPT // 2), CPT // 2)], idx_a)
        plsc.subcore_barrier()

        def body(o, carry):
            for j in range(8):
                pltpu.async_copy(
                    ones_v, acc_sh.at[idx_a.at[o * 8 + j]], sem, add=True
                )
            for j in range(8):
                pltpu.make_async_copy(
                    ones_v, acc_sh.at[idx_a.at[0]], sem
                ).wait()
            return carry

        lax.fori_loop(0, CPT // 2 // 8, body, 0)
        plsc.subcore_barrier()

        def obody(j, carry):
            off = s * RPS + j * CHUNK
            pltpu.sync_copy(acc_sh.at[pl.ds(off, CHUNK)], stage_v)
            pltpu.sync_copy(stage_v, out_hbm.at[pl.ds(c * N_PAD + off, CHUNK)])
            return carry

        lax.fori_loop(0, RPS // CHUNK, obody, 0)

    return k(dst3, onz)


def _sc_agg(g_sp, src3, dst3, zeros_rows):
    @functools.partial(
        pl.kernel,
        mesh=_mesh(),
        compiler_params=_sc_params,
        out_type=jax.ShapeDtypeStruct((NC, N_PAD, HALF), jnp.float32),
        scratch_types=[
            pltpu.VMEM((CPT, CHUNK), jnp.int32),
            pltpu.VMEM((CPT, CHUNK), jnp.int32),
            [pltpu.VMEM((CHUNK, HALF), jnp.float32)] * NBUF,
            [pltpu.SemaphoreType.DMA] * NBUF,
            [pltpu.SemaphoreType.DMA] * NBUF,
            pltpu.VMEM_SHARED((N_PAD, HALF), jnp.float32),
            pltpu.VMEM_SHARED((N_PAD, HALF), jnp.float32),
        ],
    )
    def k(g_hbm, src_hbm, dst_hbm, z_hbm, out_hbm,
          src_a, dst_a, rows, gsem, ssem, acc_sh, g_sh):
        c = lax.axis_index("c")
        s = lax.axis_index("s")

        pltpu.sync_copy(z_hbm, rows[0])

        def zbody(j, carry):
            off = s * RPS + j * CHUNK
            pltpu.sync_copy(rows[0], acc_sh.at[pl.ds(off, CHUNK)])
            pltpu.sync_copy(g_hbm.at[c].at[pl.ds(off, CHUNK)], rows[1])
            pltpu.sync_copy(rows[1], g_sh.at[pl.ds(off, CHUNK)])
            return carry

        lax.fori_loop(0, RPS // CHUNK, zbody, 0)

        pltpu.sync_copy(src_hbm.at[s], src_a)
        pltpu.sync_copy(dst_hbm.at[s], dst_a)
        plsc.subcore_barrier()

        for b in range(NBUF):
            pltpu.async_copy(g_sh.at[src_a.at[b]], rows[b], gsem[b])

        def outer(o, carry):
            for b in range(NBUF):
                i = o * NBUF + b
                pltpu.make_async_copy(
                    g_sh.at[src_a.at[0]], rows[b], gsem[b]
                ).wait()
                pltpu.async_copy(
                    rows[b], acc_sh.at[dst_a.at[i]], ssem[b], add=True
                )
                pltpu.make_async_copy(
                    rows[b], acc_sh.at[dst_a.at[i]], ssem[b]
                ).wait()

                @pl.when(i + NBUF < CPT)
                def _():
                    pltpu.async_copy(
                        g_sh.at[src_a.at[i + NBUF]], rows[b], gsem[b]
                    )

            return carry

        lax.fori_loop(0, CPT // NBUF, outer, 0)
        plsc.subcore_barrier()

        def obody(j, carry):
            off = s * RPS + j * CHUNK
            pltpu.sync_copy(acc_sh.at[pl.ds(off, CHUNK)], rows[0])
            pltpu.sync_copy(rows[0], out_hbm.at[c].at[pl.ds(off, CHUNK)])
            return carry

        lax.fori_loop(0, RPS // CHUNK, obody, 0)

    return k(g_sp, src3, dst3, zeros_rows)


def _split(v):
    return jnp.stack([v[:, :HALF], v[:, HALF:]], axis=0)


def _tc1_body(degp_ref, x_ref, w1_ref, dis_ref, g1_ref):
    deg = degp_ref[:, 0:1] + degp_ref[:, 1:2] + 1.0
    dis = lax.rsqrt(deg)
    dis_ref[...] = dis
    z = jnp.dot(x_ref[...], w1_ref[...], preferred_element_type=jnp.float32)
    g1_ref[...] = _split(z * dis)


def _tc1(degp, x, w1):
    grid = N_PAD // ROW_BLK
    return pl.pallas_call(
        _tc1_body,
        grid=(grid,),
        in_specs=[
            pl.BlockSpec((ROW_BLK, 2), lambda i: (i, 0)),
            pl.BlockSpec((ROW_BLK, F_IN), lambda i: (i, 0)),
            pl.BlockSpec((F_IN, F_HID), lambda i: (0, 0)),
        ],
        out_specs=[
            pl.BlockSpec((ROW_BLK, 1), lambda i: (i, 0)),
            pl.BlockSpec((2, ROW_BLK, HALF), lambda i: (0, i, 0)),
        ],
        out_shape=[
            jax.ShapeDtypeStruct((N_PAD, 1), jnp.float32),
            jax.ShapeDtypeStruct((2, N_PAD, HALF), jnp.float32),
        ],
    )(degp, x, w1)


def _tc2_body(acc_ref, g1_ref, dis_ref, b1_ref, w2_ref, g2_ref):
    dis = dis_ref[...]
    pre = (acc_ref[...] + g1_ref[...]) * dis[None] + b1_ref[...]
    h1 = jnp.maximum(jnp.concatenate([pre[0], pre[1]], axis=1), 0.0)
    z2 = jnp.dot(h1, w2_ref[...], preferred_element_type=jnp.float32)
    g2_ref[...] = _split(z2 * dis)


def _tc2(acc, g1, dis, b1, w2):
    grid = N_PAD // ROW_BLK
    sb = pl.BlockSpec((2, ROW_BLK, HALF), lambda i: (0, i, 0))
    return pl.pallas_call(
        _tc2_body,
        grid=(grid,),
        in_specs=[
            sb,
            sb,
            pl.BlockSpec((ROW_BLK, 1), lambda i: (i, 0)),
            pl.BlockSpec((2, 1, HALF), lambda i: (0, 0, 0)),
            pl.BlockSpec((F_HID, F_HID), lambda i: (0, 0)),
        ],
        out_specs=sb,
        out_shape=jax.ShapeDtypeStruct((2, N_PAD, HALF), jnp.float32),
    )(acc, g1, dis, b1, w2)


def _tc3_body(
    acc_ref, g2_ref, dis_ref, b2_ref, batch_ref, wl_ref, bl_ref,
    out_ref, pooled_ref, cnt_ref,
):
    i = pl.program_id(0)

    @pl.when(i == 0)
    def _():
        pooled_ref[...] = jnp.zeros_like(pooled_ref)
        cnt_ref[...] = jnp.zeros_like(cnt_ref)

    pre = (acc_ref[...] + g2_ref[...]) * dis_ref[...][None] + b2_ref[...]
    h2 = jnp.maximum(jnp.concatenate([pre[0], pre[1]], axis=1), 0.0)
    ids = batch_ref[...]
    onehot = (
        ids == lax.broadcasted_iota(jnp.int32, (1, N_GRAPHS), 1)
    ).astype(jnp.float32)
    dn = (((0,), (0,)), ((), ()))
    pooled_ref[...] += lax.dot_general(
        onehot, h2, dn, preferred_element_type=jnp.float32
    )
    cnt_ref[...] += lax.dot_general(
        onehot,
        jnp.ones((ROW_BLK, 1), jnp.float32),
        dn,
        preferred_element_type=jnp.float32,
    )

    @pl.when(i == pl.num_programs(0) - 1)
    def _():
        mean = pooled_ref[...] / jnp.maximum(cnt_ref[...], 1.0)
        out_ref[...] = (
            jnp.dot(mean, wl_ref[...], preferred_element_type=jnp.float32)
            + bl_ref[...]
        )


def _tc3(acc, g2, dis, b2, batchp, wl, bl):
    grid = N_PAD // ROW_BLK
    sb = pl.BlockSpec((2, ROW_BLK, HALF), lambda i: (0, i, 0))
    return pl.pallas_call(
        _tc3_body,
        grid=(grid,),
        in_specs=[
            sb,
            sb,
            pl.BlockSpec((ROW_BLK, 1), lambda i: (i, 0)),
            pl.BlockSpec((2, 1, HALF), lambda i: (0, 0, 0)),
            pl.BlockSpec((ROW_BLK, 1), lambda i: (i, 0)),
            pl.BlockSpec((F_HID, N_CLS), lambda i: (0, 0)),
            pl.BlockSpec((1, N_CLS), lambda i: (0, 0)),
        ],
        out_specs=pl.BlockSpec((N_GRAPHS, N_CLS), lambda i: (0, 0)),
        out_shape=jax.ShapeDtypeStruct((N_GRAPHS, N_CLS), jnp.float32),
        scratch_shapes=[
            pltpu.VMEM((N_GRAPHS, N_GRAPHS), jnp.float32),
            pltpu.VMEM((N_GRAPHS, 1), jnp.float32),
        ],
    )(acc, g2, dis, b2, batchp, wl, bl)


def kernel(x, edge_index, batch, W1, b1, W2, b2, Wl, bl):
    pad_idx = N_PAD - 1
    src3 = jnp.pad(
        edge_index[0], (0, E_PAD - N_EDGES), constant_values=pad_idx
    ).reshape(NS, CPT, CHUNK)
    dst3 = jnp.pad(
        edge_index[1], (0, E_PAD - N_EDGES), constant_values=pad_idx
    ).reshape(NS, CPT, CHUNK)

    x_p = jnp.pad(x, ((0, N_PAD - N_NODES), (0, 0)))
    batch_p = jnp.pad(
        batch, (0, N_PAD - N_NODES), constant_values=N_GRAPHS
    ).reshape(N_PAD, 1)

    onz = jnp.concatenate(
        [jnp.ones((CHUNK,), jnp.float32), jnp.zeros((CHUNK,), jnp.float32)]
    )
    zeros_rows = jnp.zeros((CHUNK, HALF), jnp.float32)

    deg_flat = _sc_degree(dst3, onz)
    degp = deg_flat.reshape(NC, N_PAD).T

    dis, g1 = _tc1(degp, x_p, W1)

    acc1 = _sc_agg(g1, src3, dst3, zeros_rows)
    g2 = _tc2(acc1, g1, dis, _split(b1.reshape(1, F_HID)), W2)

    acc2 = _sc_agg(g2, src3, dst3, zeros_rows)
    out = _tc3(
        acc2, g2, dis, _split(b2.reshape(1, F_HID)), batch_p, Wl,
        bl.reshape(1, N_CLS),
    )
    return out


# speedup vs baseline: 2.3475x; 1.0161x over previous
import functools

import jax
import jax.numpy as jnp
from jax import lax
from jax.experimental import pallas as pl
from jax.experimental.pallas import tpu as pltpu
from jax.experimental.pallas import tpu_sc as plsc

N_NODES = 10000
N_EDGES = 320000
F_IN = 128
F_HID = 64
HALF = F_HID // 2
N_CLS = 3
N_GRAPHS = 64

NC = 2
NS = 16
N_PAD = 10240
PAD_IDX = N_PAD - 1
RPS = N_PAD // NS
CHUNK = 128
EPT = N_EDGES // NS
CPT = 160
EPC = N_EDGES // NS // NC
DPC = 80
NBUF = 4
ROW_BLK = 1024

_mesh = functools.partial(
    plsc.VectorSubcoreMesh, core_axis_name="c", subcore_axis_name="s"
)
_sc_params = pltpu.CompilerParams(use_tc_tiling_on_sc=False)


def _fill_pad(ref, start, count):
    def body(k, carry):
        ref[pl.ds(start + k * 16, 16)] = jnp.full((16,), PAD_IDX, jnp.int32)
        return carry

    lax.fori_loop(0, count, body, 0)


def _sc_degree(dst2, onz):
    @functools.partial(
        pl.kernel,
        mesh=_mesh(),
        compiler_params=_sc_params,
        out_type=jax.ShapeDtypeStruct((NC * N_PAD,), jnp.float32),
        scratch_types=[
            pltpu.VMEM((DPC * CHUNK,), jnp.int32),
            pltpu.VMEM((CHUNK,), jnp.float32),
            pltpu.VMEM((CHUNK,), jnp.float32),
            pltpu.VMEM_SHARED((N_PAD,), jnp.float32),
            pltpu.SemaphoreType.DMA,
        ],
    )
    def k(dst_hbm, onz_hbm, out_hbm, idx_a, ones_v, stage_v, acc_sh, sem):
        c = lax.axis_index("c")
        s = lax.axis_index("s")
        pltpu.sync_copy(onz_hbm.at[pl.ds(0, CHUNK)], ones_v)
        pltpu.sync_copy(onz_hbm.at[pl.ds(CHUNK, CHUNK)], stage_v)

        def zbody(j, carry):
            pltpu.sync_copy(
                stage_v, acc_sh.at[pl.ds(s * RPS + j * CHUNK, CHUNK)]
            )
            return carry

        lax.fori_loop(0, RPS // CHUNK, zbody, 0)
        pltpu.sync_copy(
            dst_hbm.at[s].at[pl.ds(c * EPC, EPC)], idx_a.at[pl.ds(0, EPC)]
        )
        _fill_pad(idx_a, EPC, (DPC * CHUNK - EPC) // 16)
        plsc.subcore_barrier()

        def body(o, carry):
            for j in range(8):
                pltpu.async_copy(
                    ones_v,
                    acc_sh.at[idx_a.at[pl.ds((o * 8 + j) * CHUNK, CHUNK)]],
                    sem,
                    add=True,
                )
            for j in range(8):
                pltpu.make_async_copy(
                    ones_v, acc_sh.at[idx_a.at[pl.ds(0, CHUNK)]], sem
                ).wait()
            return carry

        lax.fori_loop(0, DPC // 8, body, 0)
        plsc.subcore_barrier()

        def obody(j, carry):
            off = s * RPS + j * CHUNK
            pltpu.sync_copy(acc_sh.at[pl.ds(off, CHUNK)], stage_v)
            pltpu.sync_copy(stage_v, out_hbm.at[pl.ds(c * N_PAD + off, CHUNK)])
            return carry

        lax.fori_loop(0, RPS // CHUNK, obody, 0)

    return k(dst2, onz)


def _sc_agg(g_sp, src2, dst2, zeros_rows):
    @functools.partial(
        pl.kernel,
        mesh=_mesh(),
        compiler_params=_sc_params,
        out_type=jax.ShapeDtypeStruct((NC, N_PAD, HALF), jnp.float32),
        scratch_types=[
            pltpu.VMEM((CPT * CHUNK,), jnp.int32),
            pltpu.VMEM((CPT * CHUNK,), jnp.int32),
            [pltpu.VMEM((CHUNK, HALF), jnp.float32)] * NBUF,
            [pltpu.SemaphoreType.DMA] * NBUF,
            [pltpu.SemaphoreType.DMA] * NBUF,
            pltpu.VMEM_SHARED((N_PAD, HALF), jnp.float32),
            pltpu.VMEM_SHARED((N_PAD, HALF), jnp.float32),
        ],
    )
    def k(g_hbm, src_hbm, dst_hbm, z_hbm, out_hbm,
          src_a, dst_a, rows, gsem, ssem, acc_sh, g_sh):
        c = lax.axis_index("c")
        s = lax.axis_index("s")

        pltpu.sync_copy(z_hbm, rows[0])

        def zbody(j, carry):
            off = s * RPS + j * CHUNK
            pltpu.sync_copy(rows[0], acc_sh.at[pl.ds(off, CHUNK)])
            pltpu.sync_copy(g_hbm.at[c].at[pl.ds(off, CHUNK)], rows[1])
            pltpu.sync_copy(rows[1], g_sh.at[pl.ds(off, CHUNK)])
            return carry

        lax.fori_loop(0, RPS // CHUNK, zbody, 0)

        pltpu.sync_copy(src_hbm.at[s], src_a.at[pl.ds(0, EPT)])
        pltpu.sync_copy(dst_hbm.at[s], dst_a.at[pl.ds(0, EPT)])
        _fill_pad(src_a, EPT, (CPT * CHUNK - EPT) // 16)
        _fill_pad(dst_a, EPT, (CPT * CHUNK - EPT) // 16)
        plsc.subcore_barrier()

        for b in range(NBUF):
            pltpu.async_copy(
                g_sh.at[src_a.at[pl.ds(b * CHUNK, CHUNK)]], rows[b], gsem[b]
            )

        def outer(o, carry):
            for b in range(NBUF):
                i = o * NBUF + b
                pltpu.make_async_copy(
                    g_sh.at[src_a.at[pl.ds(0, CHUNK)]], rows[b], gsem[b]
                ).wait()
                pltpu.async_copy(
                    rows[b],
                    acc_sh.at[dst_a.at[pl.ds(i * CHUNK, CHUNK)]],
                    ssem[b],
                    add=True,
                )
                pltpu.make_async_copy(
                    rows[b],
                    acc_sh.at[dst_a.at[pl.ds(i * CHUNK, CHUNK)]],
                    ssem[b],
                ).wait()

                @pl.when(i + NBUF < CPT)
                def _():
                    pltpu.async_copy(
                        g_sh.at[src_a.at[pl.ds((i + NBUF) * CHUNK, CHUNK)]],
                        rows[b],
                        gsem[b],
                    )

            return carry

        lax.fori_loop(0, CPT // NBUF, outer, 0)
        plsc.subcore_barrier()

        def obody(j, carry):
            off = s * RPS + j * CHUNK
            pltpu.sync_copy(acc_sh.at[pl.ds(off, CHUNK)], rows[0])
            pltpu.sync_copy(rows[0], out_hbm.at[c].at[pl.ds(off, CHUNK)])
            return carry

        lax.fori_loop(0, RPS // CHUNK, obody, 0)

    return k(g_sp, src2, dst2, zeros_rows)


def _split(v):
    return jnp.stack([v[:, :HALF], v[:, HALF:]], axis=0)


def _tcz_body(x_ref, w1_ref, z_ref):
    z_ref[...] = jnp.dot(
        x_ref[...], w1_ref[...], preferred_element_type=jnp.float32
    )


def _tcz(x, w1):
    grid = N_PAD // ROW_BLK
    return pl.pallas_call(
        _tcz_body,
        grid=(grid,),
        in_specs=[
            pl.BlockSpec((ROW_BLK, F_IN), lambda i: (i, 0)),
            pl.BlockSpec((F_IN, F_HID), lambda i: (0, 0)),
        ],
        out_specs=pl.BlockSpec((ROW_BLK, F_HID), lambda i: (i, 0)),
        out_shape=jax.ShapeDtypeStruct((N_PAD, F_HID), jnp.float32),
    )(x, w1)


def _tc1_body(degp_ref, z_ref, dis_ref, g1_ref):
    deg = degp_ref[:, 0:1] + degp_ref[:, 1:2] + 1.0
    dis = lax.rsqrt(deg)
    dis_ref[...] = dis
    g1_ref[...] = _split(z_ref[...] * dis)


def _tc1(degp, z):
    grid = N_PAD // ROW_BLK
    return pl.pallas_call(
        _tc1_body,
        grid=(grid,),
        in_specs=[
            pl.BlockSpec((ROW_BLK, 2), lambda i: (i, 0)),
            pl.BlockSpec((ROW_BLK, F_HID), lambda i: (i, 0)),
        ],
        out_specs=[
            pl.BlockSpec((ROW_BLK, 1), lambda i: (i, 0)),
            pl.BlockSpec((2, ROW_BLK, HALF), lambda i: (0, i, 0)),
        ],
        out_shape=[
            jax.ShapeDtypeStruct((N_PAD, 1), jnp.float32),
            jax.ShapeDtypeStruct((2, N_PAD, HALF), jnp.float32),
        ],
    )(degp, z)


def _tc2_body(acc_ref, g1_ref, dis_ref, b1_ref, w2_ref, g2_ref):
    dis = dis_ref[...]
    pre = (acc_ref[...] + g1_ref[...]) * dis[None] + b1_ref[...]
    h1 = jnp.maximum(jnp.concatenate([pre[0], pre[1]], axis=1), 0.0)
    z2 = jnp.dot(h1, w2_ref[...], preferred_element_type=jnp.float32)
    g2_ref[...] = _split(z2 * dis)


def _tc2(acc, g1, dis, b1, w2):
    grid = N_PAD // ROW_BLK
    sb = pl.BlockSpec((2, ROW_BLK, HALF), lambda i: (0, i, 0))
    return pl.pallas_call(
        _tc2_body,
        grid=(grid,),
        in_specs=[
            sb,
            sb,
            pl.BlockSpec((ROW_BLK, 1), lambda i: (i, 0)),
            pl.BlockSpec((2, 1, HALF), lambda i: (0, 0, 0)),
            pl.BlockSpec((F_HID, F_HID), lambda i: (0, 0)),
        ],
        out_specs=sb,
        out_shape=jax.ShapeDtypeStruct((2, N_PAD, HALF), jnp.float32),
    )(acc, g1, dis, b1, w2)


def _tc3_body(
    acc_ref, g2_ref, dis_ref, b2_ref, batch_ref, wl_ref, bl_ref,
    out_ref, pooled_ref, cnt_ref,
):
    i = pl.program_id(0)

    @pl.when(i == 0)
    def _():
        pooled_ref[...] = jnp.zeros_like(pooled_ref)
        cnt_ref[...] = jnp.zeros_like(cnt_ref)

    pre = (acc_ref[...] + g2_ref[...]) * dis_ref[...][None] + b2_ref[...]
    h2 = jnp.maximum(jnp.concatenate([pre[0], pre[1]], axis=1), 0.0)
    ids = batch_ref[...]
    onehot = (
        ids == lax.broadcasted_iota(jnp.int32, (1, N_GRAPHS), 1)
    ).astype(jnp.float32)
    dn = (((0,), (0,)), ((), ()))
    pooled_ref[...] += lax.dot_general(
        onehot, h2, dn, preferred_element_type=jnp.float32
    )
    cnt_ref[...] += lax.dot_general(
        onehot,
        jnp.ones((ROW_BLK, 1), jnp.float32),
        dn,
        preferred_element_type=jnp.float32,
    )

    @pl.when(i == pl.num_programs(0) - 1)
    def _():
        mean = pooled_ref[...] / jnp.maximum(cnt_ref[...], 1.0)
        out_ref[...] = (
            jnp.dot(mean, wl_ref[...], preferred_element_type=jnp.float32)
            + bl_ref[...]
        )


def _tc3(acc, g2, dis, b2, batchp, wl, bl):
    grid = N_PAD // ROW_BLK
    sb = pl.BlockSpec((2, ROW_BLK, HALF), lambda i: (0, i, 0))
    return pl.pallas_call(
        _tc3_body,
        grid=(grid,),
        in_specs=[
            sb,
            sb,
            pl.BlockSpec((ROW_BLK, 1), lambda i: (i, 0)),
            pl.BlockSpec((2, 1, HALF), lambda i: (0, 0, 0)),
            pl.BlockSpec((ROW_BLK, 1), lambda i: (i, 0)),
            pl.BlockSpec((F_HID, N_CLS), lambda i: (0, 0)),
            pl.BlockSpec((1, N_CLS), lambda i: (0, 0)),
        ],
        out_specs=pl.BlockSpec((N_GRAPHS, N_CLS), lambda i: (0, 0)),
        out_shape=jax.ShapeDtypeStruct((N_GRAPHS, N_CLS), jnp.float32),
        scratch_shapes=[
            pltpu.VMEM((N_GRAPHS, N_GRAPHS), jnp.float32),
            pltpu.VMEM((N_GRAPHS, 1), jnp.float32),
        ],
    )(acc, g2, dis, b2, batchp, wl, bl)


def kernel(x, edge_index, batch, W1, b1, W2, b2, Wl, bl):
    src2 = edge_index[0].reshape(NS, EPT)
    dst2 = edge_index[1].reshape(NS, EPT)

    x_p = jnp.pad(x, ((0, N_PAD - N_NODES), (0, 0)))
    batch_p = jnp.pad(
        batch, (0, N_PAD - N_NODES), constant_values=N_GRAPHS
    ).reshape(N_PAD, 1)

    onz = jnp.concatenate(
        [jnp.ones((CHUNK,), jnp.float32), jnp.zeros((CHUNK,), jnp.float32)]
    )
    zeros_rows = jnp.zeros((CHUNK, HALF), jnp.float32)

    z1 = _tcz(x_p, W1)
    deg_flat = _sc_degree(dst2, onz)
    degp = deg_flat.reshape(NC, N_PAD).T

    dis, g1 = _tc1(degp, z1)

    acc1 = _sc_agg(g1, src2, dst2, zeros_rows)
    g2 = _tc2(acc1, g1, dis, _split(b1.reshape(1, F_HID)), W2)

    acc2 = _sc_agg(g2, src2, dst2, zeros_rows)
    out = _tc3(
        acc2, g2, dis, _split(b2.reshape(1, F_HID)), batch_p, Wl,
        bl.reshape(1, N_CLS),
    )
    return out


# speedup vs baseline: 2.5140x; 1.0709x over previous
import functools

import jax
import jax.numpy as jnp
from jax import lax
from jax.experimental import pallas as pl
from jax.experimental.pallas import tpu as pltpu
from jax.experimental.pallas import tpu_sc as plsc

N_NODES = 10000
N_EDGES = 320000
F_IN = 128
F_HID = 64
HALF = F_HID // 2
N_CLS = 3
N_GRAPHS = 64

NC = 2
NS = 16
N_PAD = 10240
PAD_IDX = N_PAD - 1
RPS = N_PAD // NS
CHUNK = 128
EPT = N_EDGES // NS
CPT = 160
EPC = N_EDGES // NS // NC
DPC = 80
NBUF = 4
ROW_BLK = 1024

_mesh = functools.partial(
    plsc.VectorSubcoreMesh, core_axis_name="c", subcore_axis_name="s"
)
_sc_params = pltpu.CompilerParams(use_tc_tiling_on_sc=False)


def _fill_pad(ref, start, count):
    def body(k, carry):
        ref[pl.ds(start + k * 16, 16)] = jnp.full((16,), PAD_IDX, jnp.int32)
        return carry

    lax.fori_loop(0, count, body, 0)


def _sc_degree(dst2, onz):
    @functools.partial(
        pl.kernel,
        mesh=_mesh(),
        compiler_params=_sc_params,
        out_type=jax.ShapeDtypeStruct((NC * N_PAD,), jnp.float32),
        scratch_types=[
            pltpu.VMEM((DPC * CHUNK,), jnp.int32),
            pltpu.VMEM((CHUNK,), jnp.float32),
            pltpu.VMEM((CHUNK,), jnp.float32),
            pltpu.VMEM_SHARED((N_PAD,), jnp.float32),
            pltpu.SemaphoreType.DMA,
        ],
    )
    def k(dst_hbm, onz_hbm, out_hbm, idx_a, ones_v, stage_v, acc_sh, sem):
        c = lax.axis_index("c")
        s = lax.axis_index("s")
        pltpu.sync_copy(onz_hbm.at[pl.ds(0, CHUNK)], ones_v)
        pltpu.sync_copy(onz_hbm.at[pl.ds(CHUNK, CHUNK)], stage_v)

        def zbody(j, carry):
            pltpu.sync_copy(
                stage_v, acc_sh.at[pl.ds(s * RPS + j * CHUNK, CHUNK)]
            )
            return carry

        lax.fori_loop(0, RPS // CHUNK, zbody, 0)
        pltpu.sync_copy(
            dst_hbm.at[s].at[pl.ds(c * EPC, EPC)], idx_a.at[pl.ds(0, EPC)]
        )
        _fill_pad(idx_a, EPC, (DPC * CHUNK - EPC) // 16)
        plsc.subcore_barrier()

        def body(o, carry):
            for j in range(8):
                pltpu.async_copy(
                    ones_v,
                    acc_sh.at[idx_a.at[pl.ds((o * 8 + j) * CHUNK, CHUNK)]],
                    sem,
                    add=True,
                )
            for j in range(8):
                pltpu.make_async_copy(
                    ones_v, acc_sh.at[idx_a.at[pl.ds(0, CHUNK)]], sem
                ).wait()
            return carry

        lax.fori_loop(0, DPC // 8, body, 0)
        plsc.subcore_barrier()

        def obody(j, carry):
            off = s * RPS + j * CHUNK
            pltpu.sync_copy(acc_sh.at[pl.ds(off, CHUNK)], stage_v)
            pltpu.sync_copy(stage_v, out_hbm.at[pl.ds(c * N_PAD + off, CHUNK)])
            return carry

        lax.fori_loop(0, RPS // CHUNK, obody, 0)

    return k(dst2, onz)


def _sc_agg(g_sp, src2, dst2, zeros_rows):
    @functools.partial(
        pl.kernel,
        mesh=_mesh(),
        compiler_params=_sc_params,
        out_type=jax.ShapeDtypeStruct((NC, N_PAD, HALF), jnp.float32),
        scratch_types=[
            pltpu.VMEM((CPT * CHUNK,), jnp.int32),
            pltpu.VMEM((CPT * CHUNK,), jnp.int32),
            [pltpu.VMEM((CHUNK, HALF), jnp.float32)] * NBUF,
            [pltpu.SemaphoreType.DMA] * NBUF,
            [pltpu.SemaphoreType.DMA] * NBUF,
            pltpu.VMEM_SHARED((N_PAD, HALF), jnp.float32),
            pltpu.VMEM_SHARED((N_PAD, HALF), jnp.float32),
        ],
    )
    def k(g_hbm, src_hbm, dst_hbm, z_hbm, out_hbm,
          src_a, dst_a, rows, gsem, ssem, acc_sh, g_sh):
        c = lax.axis_index("c")
        s = lax.axis_index("s")

        nz = RPS // CHUNK
        sl = lambda j: pl.ds(s * RPS + j * CHUNK, CHUNK)
        pltpu.sync_copy(z_hbm, rows[0])
        pltpu.async_copy(src_hbm.at[s], src_a.at[pl.ds(0, EPT)], gsem[0])
        pltpu.async_copy(dst_hbm.at[s], dst_a.at[pl.ds(0, EPT)], gsem[0])
        for j in range(nz):
            pltpu.async_copy(rows[0], acc_sh.at[sl(j)], ssem[0])
        for j in range(nz):
            b = 1 + (j % 3)
            if j >= 3:
                pltpu.make_async_copy(
                    rows[b], g_sh.at[sl(j - 3)], ssem[b]
                ).wait()
            pltpu.async_copy(g_hbm.at[c].at[sl(j)], rows[b], gsem[b])
            pltpu.make_async_copy(
                g_hbm.at[c].at[sl(j)], rows[b], gsem[b]
            ).wait()
            pltpu.async_copy(rows[b], g_sh.at[sl(j)], ssem[b])
        for j in range(nz - 3, nz):
            b = 1 + (j % 3)
            pltpu.make_async_copy(rows[b], g_sh.at[sl(j)], ssem[b]).wait()
        for j in range(nz):
            pltpu.make_async_copy(rows[0], acc_sh.at[sl(j)], ssem[0]).wait()
        pltpu.make_async_copy(
            src_hbm.at[s], src_a.at[pl.ds(0, EPT)], gsem[0]
        ).wait()
        pltpu.make_async_copy(
            dst_hbm.at[s], dst_a.at[pl.ds(0, EPT)], gsem[0]
        ).wait()
        _fill_pad(src_a, EPT, (CPT * CHUNK - EPT) // 16)
        _fill_pad(dst_a, EPT, (CPT * CHUNK - EPT) // 16)
        plsc.subcore_barrier()

        for b in range(NBUF):
            pltpu.async_copy(
                g_sh.at[src_a.at[pl.ds(b * CHUNK, CHUNK)]], rows[b], gsem[b]
            )

        def outer(o, carry):
            for b in range(NBUF):
                i = o * NBUF + b
                pltpu.make_async_copy(
                    g_sh.at[src_a.at[pl.ds(0, CHUNK)]], rows[b], gsem[b]
                ).wait()
                pltpu.async_copy(
                    rows[b],
                    acc_sh.at[dst_a.at[pl.ds(i * CHUNK, CHUNK)]],
                    ssem[b],
                    add=True,
                )
                pltpu.make_async_copy(
                    rows[b],
                    acc_sh.at[dst_a.at[pl.ds(i * CHUNK, CHUNK)]],
                    ssem[b],
                ).wait()

                @pl.when(i + NBUF < CPT)
                def _():
                    pltpu.async_copy(
                        g_sh.at[src_a.at[pl.ds((i + NBUF) * CHUNK, CHUNK)]],
                        rows[b],
                        gsem[b],
                    )

            return carry

        lax.fori_loop(0, CPT // NBUF, outer, 0)
        plsc.subcore_barrier()

        def obody(j, carry):
            off = s * RPS + j * CHUNK
            pltpu.sync_copy(acc_sh.at[pl.ds(off, CHUNK)], rows[0])
            pltpu.sync_copy(rows[0], out_hbm.at[c].at[pl.ds(off, CHUNK)])
            return carry

        lax.fori_loop(0, RPS // CHUNK, obody, 0)

    return k(g_sp, src2, dst2, zeros_rows)


def _split(v):
    return jnp.stack([v[:, :HALF], v[:, HALF:]], axis=0)


def _pack(v):
    return v.reshape(2, v.shape[1] // 4, 128)


def _prep_body(ei_ref, s_ref, d_ref):
    s_ref[...] = ei_ref[0, :]
    d_ref[...] = ei_ref[1, :]


def _prep(edge_index):
    ob = pl.BlockSpec((N_EDGES,), lambda: (0,))
    return pl.pallas_call(
        _prep_body,
        in_specs=[pl.BlockSpec((2, N_EDGES), lambda: (0, 0))],
        out_specs=[ob, ob],
        out_shape=[
            jax.ShapeDtypeStruct((N_EDGES,), jnp.int32),
            jax.ShapeDtypeStruct((N_EDGES,), jnp.int32),
        ],
    )(edge_index)


def _tcz_body(x_ref, w1_ref, z_ref):
    z_ref[...] = jnp.dot(
        x_ref[...], w1_ref[...], preferred_element_type=jnp.float32
    )


def _tcz(x, w1):
    grid = N_PAD // ROW_BLK
    return pl.pallas_call(
        _tcz_body,
        grid=(grid,),
        in_specs=[
            pl.BlockSpec((ROW_BLK, F_IN), lambda i: (i, 0)),
            pl.BlockSpec((F_IN, F_HID), lambda i: (0, 0)),
        ],
        out_specs=pl.BlockSpec((ROW_BLK, F_HID), lambda i: (i, 0)),
        out_shape=jax.ShapeDtypeStruct((N_PAD, F_HID), jnp.float32),
    )(x, w1)


def _tc1_body(degp_ref, z_ref, dis_ref, g1_ref):
    deg = degp_ref[:, 0:1] + degp_ref[:, 1:2] + 1.0
    dis = lax.rsqrt(deg)
    dis_ref[...] = dis
    g1_ref[...] = _split(z_ref[...] * dis)


def _tc1(degp, z):
    grid = N_PAD // ROW_BLK
    return pl.pallas_call(
        _tc1_body,
        grid=(grid,),
        in_specs=[
            pl.BlockSpec((ROW_BLK, 2), lambda i: (i, 0)),
            pl.BlockSpec((ROW_BLK, F_HID), lambda i: (i, 0)),
        ],
        out_specs=[
            pl.BlockSpec((ROW_BLK, 1), lambda i: (i, 0)),
            pl.BlockSpec((2, ROW_BLK, HALF), lambda i: (0, i, 0)),
        ],
        out_shape=[
            jax.ShapeDtypeStruct((N_PAD, 1), jnp.float32),
            jax.ShapeDtypeStruct((2, N_PAD, HALF), jnp.float32),
        ],
    )(degp, z)


def _tc2_body(acc_ref, g1_ref, dis_ref, b1_ref, w2_ref, g2_ref):
    dis = dis_ref[...]
    pre = (acc_ref[...] + g1_ref[...]) * dis[None] + b1_ref[...]
    h1 = jnp.maximum(jnp.concatenate([pre[0], pre[1]], axis=1), 0.0)
    z2 = jnp.dot(h1, w2_ref[...], preferred_element_type=jnp.float32)
    g2_ref[...] = _split(z2 * dis)


def _tc2(acc, g1, dis, b1, w2):
    grid = N_PAD // ROW_BLK
    sb = pl.BlockSpec((2, ROW_BLK, HALF), lambda i: (0, i, 0))
    return pl.pallas_call(
        _tc2_body,
        grid=(grid,),
        in_specs=[
            sb,
            sb,
            pl.BlockSpec((ROW_BLK, 1), lambda i: (i, 0)),
            pl.BlockSpec((2, 1, HALF), lambda i: (0, 0, 0)),
            pl.BlockSpec((F_HID, F_HID), lambda i: (0, 0)),
        ],
        out_specs=sb,
        out_shape=jax.ShapeDtypeStruct((2, N_PAD, HALF), jnp.float32),
    )(acc, g1, dis, b1, w2)


def _tc3_body(
    acc_ref, g2_ref, dis_ref, b2_ref, batch_ref, wl_ref, bl_ref,
    out_ref, pooled_ref, cnt_ref,
):
    i = pl.program_id(0)

    @pl.when(i == 0)
    def _():
        pooled_ref[...] = jnp.zeros_like(pooled_ref)
        cnt_ref[...] = jnp.zeros_like(cnt_ref)

    pre = (acc_ref[...] + g2_ref[...]) * dis_ref[...][None] + b2_ref[...]
    h2 = jnp.maximum(jnp.concatenate([pre[0], pre[1]], axis=1), 0.0)
    ids = batch_ref[...]
    onehot = (
        ids == lax.broadcasted_iota(jnp.int32, (1, N_GRAPHS), 1)
    ).astype(jnp.float32)
    dn = (((0,), (0,)), ((), ()))
    pooled_ref[...] += lax.dot_general(
        onehot, h2, dn, preferred_element_type=jnp.float32
    )
    cnt_ref[...] += lax.dot_general(
        onehot,
        jnp.ones((ROW_BLK, 1), jnp.float32),
        dn,
        preferred_element_type=jnp.float32,
    )

    @pl.when(i == pl.num_programs(0) - 1)
    def _():
        mean = pooled_ref[...] / jnp.maximum(cnt_ref[...], 1.0)
        out_ref[...] = (
            jnp.dot(mean, wl_ref[...], preferred_element_type=jnp.float32)
            + bl_ref[...]
        )


def _tc3(acc, g2, dis, b2, batchp, wl, bl):
    grid = N_PAD // ROW_BLK
    sb = pl.BlockSpec((2, ROW_BLK, HALF), lambda i: (0, i, 0))
    return pl.pallas_call(
        _tc3_body,
        grid=(grid,),
        in_specs=[
            sb,
            sb,
            pl.BlockSpec((ROW_BLK, 1), lambda i: (i, 0)),
            pl.BlockSpec((2, 1, HALF), lambda i: (0, 0, 0)),
            pl.BlockSpec((ROW_BLK, 1), lambda i: (i, 0)),
            pl.BlockSpec((F_HID, N_CLS), lambda i: (0, 0)),
            pl.BlockSpec((1, N_CLS), lambda i: (0, 0)),
        ],
        out_specs=pl.BlockSpec((N_GRAPHS, N_CLS), lambda i: (0, 0)),
        out_shape=jax.ShapeDtypeStruct((N_GRAPHS, N_CLS), jnp.float32),
        scratch_shapes=[
            pltpu.VMEM((N_GRAPHS, N_GRAPHS), jnp.float32),
            pltpu.VMEM((N_GRAPHS, 1), jnp.float32),
        ],
    )(acc, g2, dis, b2, batchp, wl, bl)


def kernel(x, edge_index, batch, W1, b1, W2, b2, Wl, bl):
    src_f, dst_f = _prep(edge_index)
    src2 = src_f.reshape(NS, EPT)
    dst2 = dst_f.reshape(NS, EPT)

    x_p = jnp.pad(x, ((0, N_PAD - N_NODES), (0, 0)))
    batch_p = jnp.pad(
        batch, (0, N_PAD - N_NODES), constant_values=N_GRAPHS
    ).reshape(N_PAD, 1)

    onz = jnp.concatenate(
        [jnp.ones((CHUNK,), jnp.float32), jnp.zeros((CHUNK,), jnp.float32)]
    )
    zeros_rows = jnp.zeros((CHUNK, HALF), jnp.float32)

    z1 = _tcz(x_p, W1)
    deg_flat = _sc_degree(dst2, onz)
    degp = deg_flat.reshape(NC, N_PAD).T

    dis, g1 = _tc1(degp, z1)

    acc1 = _sc_agg(g1, src2, dst2, zeros_rows)
    g2 = _tc2(acc1, g1, dis, _split(b1.reshape(1, F_HID)), W2)

    acc2 = _sc_agg(g2, src2, dst2, zeros_rows)
    out = _tc3(
        acc2, g2, dis, _split(b2.reshape(1, F_HID)), batch_p, Wl,
        bl.reshape(1, N_CLS),
    )
    return out


# speedup vs baseline: 3.0208x; 1.2016x over previous
import functools

import jax
import jax.numpy as jnp
from jax import lax
from jax.experimental import pallas as pl
from jax.experimental.pallas import tpu as pltpu
from jax.experimental.pallas import tpu_sc as plsc

N_NODES = 10000
N_EDGES = 320000
F_IN = 128
F_HID = 64
HALF = F_HID // 2
N_CLS = 3
N_GRAPHS = 64

NC = 2
NS = 16
N_PAD = 10240
PAD_IDX = N_PAD - 1
RPS = N_PAD // NS
CHUNK = 128
EPT = N_EDGES // NS
CPT = 160
EPC = N_EDGES // NS // NC
DPC = 80
NBUF = 4
ROW_BLK = 1024

_mesh = functools.partial(
    plsc.VectorSubcoreMesh, core_axis_name="c", subcore_axis_name="s"
)
_sc_params = pltpu.CompilerParams(use_tc_tiling_on_sc=False)


def _fill_pad(ref, start, count):
    def body(k, carry):
        ref[pl.ds(start + k * 16, 16)] = jnp.full((16,), PAD_IDX, jnp.int32)
        return carry

    lax.fori_loop(0, count, body, 0)


def _sc_degree(dst2, onz):
    @functools.partial(
        pl.kernel,
        mesh=_mesh(),
        compiler_params=_sc_params,
        out_type=jax.ShapeDtypeStruct((NC * N_PAD,), jnp.float32),
        scratch_types=[
            pltpu.VMEM((DPC * CHUNK,), jnp.int32),
            pltpu.VMEM((CHUNK,), jnp.float32),
            pltpu.VMEM((CHUNK,), jnp.float32),
            pltpu.VMEM_SHARED((N_PAD,), jnp.float32),
            pltpu.SemaphoreType.DMA,
        ],
    )
    def k(dst_hbm, onz_hbm, out_hbm, idx_a, ones_v, stage_v, acc_sh, sem):
        c = lax.axis_index("c")
        s = lax.axis_index("s")
        pltpu.sync_copy(onz_hbm.at[pl.ds(0, CHUNK)], ones_v)
        pltpu.sync_copy(onz_hbm.at[pl.ds(CHUNK, CHUNK)], stage_v)

        def zbody(j, carry):
            pltpu.sync_copy(
                stage_v, acc_sh.at[pl.ds(s * RPS + j * CHUNK, CHUNK)]
            )
            return carry

        lax.fori_loop(0, RPS // CHUNK, zbody, 0)
        pltpu.sync_copy(
            dst_hbm.at[s].at[pl.ds(c * EPC, EPC)], idx_a.at[pl.ds(0, EPC)]
        )
        _fill_pad(idx_a, EPC, (DPC * CHUNK - EPC) // 16)
        plsc.subcore_barrier()

        def body(o, carry):
            for j in range(8):
                pltpu.async_copy(
                    ones_v,
                    acc_sh.at[idx_a.at[pl.ds((o * 8 + j) * CHUNK, CHUNK)]],
                    sem,
                    add=True,
                )
            for j in range(8):
                pltpu.make_async_copy(
                    ones_v, acc_sh.at[idx_a.at[pl.ds(0, CHUNK)]], sem
                ).wait()
            return carry

        lax.fori_loop(0, DPC // 8, body, 0)
        plsc.subcore_barrier()

        def obody(j, carry):
            off = s * RPS + j * CHUNK
            pltpu.sync_copy(acc_sh.at[pl.ds(off, CHUNK)], stage_v)
            pltpu.sync_copy(stage_v, out_hbm.at[pl.ds(c * N_PAD + off, CHUNK)])
            return carry

        lax.fori_loop(0, RPS // CHUNK, obody, 0)

    return k(dst2, onz)


def _sc_agg(g_sp, src2, dst2, zeros_rows):
    @functools.partial(
        pl.kernel,
        mesh=_mesh(),
        compiler_params=_sc_params,
        out_type=jax.ShapeDtypeStruct((NC, N_PAD, HALF), jnp.float32),
        scratch_types=[
            pltpu.VMEM((CPT * CHUNK,), jnp.int32),
            pltpu.VMEM((CPT * CHUNK,), jnp.int32),
            [pltpu.VMEM((CHUNK, HALF), jnp.float32)] * NBUF,
            [pltpu.SemaphoreType.DMA] * NBUF,
            [pltpu.SemaphoreType.DMA] * NBUF,
            pltpu.VMEM_SHARED((N_PAD, HALF), jnp.float32),
            pltpu.VMEM_SHARED((N_PAD, HALF), jnp.float32),
        ],
    )
    def k(g_hbm, src_hbm, dst_hbm, z_hbm, out_hbm,
          src_a, dst_a, rows, gsem, ssem, acc_sh, g_sh):
        c = lax.axis_index("c")
        s = lax.axis_index("s")

        nz = RPS // CHUNK
        sl = lambda j: pl.ds(s * RPS + j * CHUNK, CHUNK)
        pltpu.sync_copy(z_hbm, rows[0])
        pltpu.async_copy(src_hbm.at[s], src_a.at[pl.ds(0, EPT)], gsem[0])
        pltpu.async_copy(dst_hbm.at[s], dst_a.at[pl.ds(0, EPT)], gsem[0])
        for j in range(nz):
            pltpu.async_copy(rows[0], acc_sh.at[sl(j)], ssem[0])
        for j in range(nz):
            b = 1 + (j % 3)
            if j >= 3:
                pltpu.make_async_copy(
                    rows[b], g_sh.at[sl(j - 3)], ssem[b]
                ).wait()
            pltpu.async_copy(g_hbm.at[c].at[sl(j)], rows[b], gsem[b])
            pltpu.make_async_copy(
                g_hbm.at[c].at[sl(j)], rows[b], gsem[b]
            ).wait()
            pltpu.async_copy(rows[b], g_sh.at[sl(j)], ssem[b])
        for j in range(nz - 3, nz):
            b = 1 + (j % 3)
            pltpu.make_async_copy(rows[b], g_sh.at[sl(j)], ssem[b]).wait()
        for j in range(nz):
            pltpu.make_async_copy(rows[0], acc_sh.at[sl(j)], ssem[0]).wait()
        pltpu.make_async_copy(
            src_hbm.at[s], src_a.at[pl.ds(0, EPT)], gsem[0]
        ).wait()
        pltpu.make_async_copy(
            dst_hbm.at[s], dst_a.at[pl.ds(0, EPT)], gsem[0]
        ).wait()
        _fill_pad(src_a, EPT, (CPT * CHUNK - EPT) // 16)
        _fill_pad(dst_a, EPT, (CPT * CHUNK - EPT) // 16)
        plsc.subcore_barrier()

        for b in range(NBUF):
            pltpu.async_copy(
                g_sh.at[src_a.at[pl.ds(b * CHUNK, CHUNK)]], rows[b], gsem[b]
            )

        def outer(o, carry):
            for b in range(NBUF):
                i = o * NBUF + b
                pltpu.make_async_copy(
                    g_sh.at[src_a.at[pl.ds(0, CHUNK)]], rows[b], gsem[b]
                ).wait()
                pltpu.async_copy(
                    rows[b],
                    acc_sh.at[dst_a.at[pl.ds(i * CHUNK, CHUNK)]],
                    ssem[b],
                    add=True,
                )
                pltpu.make_async_copy(
                    rows[b],
                    acc_sh.at[dst_a.at[pl.ds(i * CHUNK, CHUNK)]],
                    ssem[b],
                ).wait()

                @pl.when(i + NBUF < CPT)
                def _():
                    pltpu.async_copy(
                        g_sh.at[src_a.at[pl.ds((i + NBUF) * CHUNK, CHUNK)]],
                        rows[b],
                        gsem[b],
                    )

            return carry

        lax.fori_loop(0, CPT // NBUF, outer, 0)
        plsc.subcore_barrier()

        def obody(j, carry):
            off = s * RPS + j * CHUNK
            pltpu.sync_copy(acc_sh.at[pl.ds(off, CHUNK)], rows[0])
            pltpu.sync_copy(rows[0], out_hbm.at[c].at[pl.ds(off, CHUNK)])
            return carry

        lax.fori_loop(0, RPS // CHUNK, obody, 0)

    return k(g_sp, src2, dst2, zeros_rows)


def _split(v):
    return jnp.stack([v[:, :HALF], v[:, HALF:]], axis=0)


def _pack(v):
    return v.reshape(2, v.shape[1] // 4, 128)


def _prep_body(ei_ref, s_ref, d_ref):
    s_ref[...] = ei_ref[0, :]
    d_ref[...] = ei_ref[1, :]


def _prep(edge_index):
    ob = pl.BlockSpec((N_EDGES,), lambda: (0,))
    return pl.pallas_call(
        _prep_body,
        in_specs=[pl.BlockSpec((2, N_EDGES), lambda: (0, 0))],
        out_specs=[ob, ob],
        out_shape=[
            jax.ShapeDtypeStruct((N_EDGES,), jnp.int32),
            jax.ShapeDtypeStruct((N_EDGES,), jnp.int32),
        ],
    )(edge_index)


def _tcz_body(x_ref, w_ref, z_ref):
    z_ref[...] = jnp.dot(
        x_ref[...], w_ref[...], preferred_element_type=jnp.float32
    )


def _tcz(x4, w1bd):
    grid = N_PAD // ROW_BLK
    r4 = ROW_BLK // 4
    return pl.pallas_call(
        _tcz_body,
        grid=(grid,),
        in_specs=[
            pl.BlockSpec((r4, 4 * F_IN), lambda i: (i, 0)),
            pl.BlockSpec((4 * F_IN, 256), lambda i: (0, 0)),
        ],
        out_specs=pl.BlockSpec((r4, 256), lambda i: (i, 0)),
        out_shape=jax.ShapeDtypeStruct((N_PAD // 4, 256), jnp.float32),
    )(x4, w1bd)


def _lane_groups():
    row = lax.broadcasted_iota(jnp.int32, (4, 128), 0)
    lane = lax.broadcasted_iota(jnp.int32, (4, 128), 1)
    return (lane // HALF == row).astype(jnp.float32)


def _tc1_body(degq_ref, zp_ref, dp_ref, g1_ref):
    degq = degq_ref[...]
    deg4 = degq[:, 0:4] + degq[:, 4:8] + 1.0
    dis4 = lax.rsqrt(deg4)
    dp = jnp.dot(dis4, _lane_groups(), preferred_element_type=jnp.float32)
    dp_ref[...] = dp
    zp = zp_ref[...]
    g1_ref[...] = jnp.stack([zp[:, :128] * dp, zp[:, 128:] * dp], axis=0)


def _tc1(degq, zp):
    grid = N_PAD // ROW_BLK
    r4 = ROW_BLK // 4
    return pl.pallas_call(
        _tc1_body,
        grid=(grid,),
        in_specs=[
            pl.BlockSpec((r4, 8), lambda i: (i, 0)),
            pl.BlockSpec((r4, 256), lambda i: (i, 0)),
        ],
        out_specs=[
            pl.BlockSpec((r4, 128), lambda i: (i, 0)),
            pl.BlockSpec((2, r4, 128), lambda i: (0, i, 0)),
        ],
        out_shape=[
            jax.ShapeDtypeStruct((N_PAD // 4, 128), jnp.float32),
            jax.ShapeDtypeStruct((2, N_PAD // 4, 128), jnp.float32),
        ],
    )(degq, zp)


def _tc2_body(acc_ref, g1_ref, dp_ref, b1_ref, k2_ref, g2_ref):
    dp = dp_ref[...]
    pre = (acc_ref[...] + g1_ref[...]) * dp[None] + b1_ref[...]
    pre = jnp.maximum(pre, 0.0)
    h_cat = jnp.concatenate([pre[0], pre[1]], axis=1)
    z2p = jnp.dot(h_cat, k2_ref[...], preferred_element_type=jnp.float32)
    g2_ref[...] = jnp.stack([z2p[:, :128] * dp, z2p[:, 128:] * dp], axis=0)


def _tc2(acc, g1, dp, b1p, k2):
    grid = N_PAD // ROW_BLK
    r4 = ROW_BLK // 4
    sb = pl.BlockSpec((2, r4, 128), lambda i: (0, i, 0))
    return pl.pallas_call(
        _tc2_body,
        grid=(grid,),
        in_specs=[
            sb,
            sb,
            pl.BlockSpec((r4, 128), lambda i: (i, 0)),
            pl.BlockSpec((2, 1, 128), lambda i: (0, 0, 0)),
            pl.BlockSpec((256, 256), lambda i: (0, 0)),
        ],
        out_specs=sb,
        out_shape=jax.ShapeDtypeStruct((2, N_PAD // 4, 128), jnp.float32),
    )(acc, g1, dp, b1p, k2)


def _tc3_body(
    acc_ref, g2_ref, dp_ref, b2_ref, batch4_ref, wl_ref, bl_ref,
    out_ref, pooled_ref, cnt_ref,
):
    i = pl.program_id(0)

    @pl.when(i == 0)
    def _():
        pooled_ref[...] = jnp.zeros_like(pooled_ref)
        cnt_ref[...] = jnp.zeros_like(cnt_ref)

    pre = (acc_ref[...] + g2_ref[...]) * dp_ref[...][None] + b2_ref[...]
    pre = jnp.maximum(pre, 0.0)
    ids4 = batch4_ref[...]
    giota = lax.broadcasted_iota(jnp.int32, (1, N_GRAPHS), 1)
    dn = (((0,), (0,)), ((), ()))
    r4 = pre.shape[1]
    oh_sum = jnp.zeros((r4, N_GRAPHS), jnp.float32)
    for k in range(4):
        oh_k = (ids4[:, k : k + 1] == giota).astype(jnp.float32)
        oh_sum = oh_sum + oh_k
        pooled_ref[:, 0:HALF] += lax.dot_general(
            oh_k, pre[0][:, k * HALF : (k + 1) * HALF], dn,
            preferred_element_type=jnp.float32,
        )
        pooled_ref[:, HALF:F_HID] += lax.dot_general(
            oh_k, pre[1][:, k * HALF : (k + 1) * HALF], dn,
            preferred_element_type=jnp.float32,
        )
    cnt_ref[...] += lax.dot_general(
        oh_sum, jnp.ones((r4, 1), jnp.float32), dn,
        preferred_element_type=jnp.float32,
    )

    @pl.when(i == pl.num_programs(0) - 1)
    def _():
        mean = pooled_ref[...] / jnp.maximum(cnt_ref[...], 1.0)
        out_ref[...] = (
            jnp.dot(mean, wl_ref[...], preferred_element_type=jnp.float32)
            + bl_ref[...]
        )


def _tc3(acc, g2, dp, b2p, batch4, wl, bl):
    grid = N_PAD // ROW_BLK
    r4 = ROW_BLK // 4
    sb = pl.BlockSpec((2, r4, 128), lambda i: (0, i, 0))
    return pl.pallas_call(
        _tc3_body,
        grid=(grid,),
        in_specs=[
            sb,
            sb,
            pl.BlockSpec((r4, 128), lambda i: (i, 0)),
            pl.BlockSpec((2, 1, 128), lambda i: (0, 0, 0)),
            pl.BlockSpec((r4, 4), lambda i: (i, 0)),
            pl.BlockSpec((F_HID, N_CLS), lambda i: (0, 0)),
            pl.BlockSpec((1, N_CLS), lambda i: (0, 0)),
        ],
        out_specs=pl.BlockSpec((N_GRAPHS, N_CLS), lambda i: (0, 0)),
        out_shape=jax.ShapeDtypeStruct((N_GRAPHS, N_CLS), jnp.float32),
        scratch_shapes=[
            pltpu.VMEM((N_GRAPHS, F_HID), jnp.float32),
            pltpu.VMEM((N_GRAPHS, 1), jnp.float32),
        ],
    )(acc, g2, dp, b2p, batch4, wl, bl)


def _packed_weights(W1, W2):
    import numpy as np

    eye4 = jnp.eye(4, dtype=jnp.float32)
    perm = np.concatenate(
        [
            np.concatenate([np.arange(HALF) + F_HID * k for k in range(4)]),
            np.concatenate(
                [np.arange(HALF) + HALF + F_HID * k for k in range(4)]
            ),
        ]
    )
    w1bd = jnp.kron(eye4, W1)[:, perm]
    k2 = jnp.concatenate(
        [
            jnp.concatenate(
                [jnp.kron(eye4, W2[:HALF, :HALF]),
                 jnp.kron(eye4, W2[:HALF, HALF:])], axis=1
            ),
            jnp.concatenate(
                [jnp.kron(eye4, W2[HALF:, :HALF]),
                 jnp.kron(eye4, W2[HALF:, HALF:])], axis=1
            ),
        ],
        axis=0,
    )
    return w1bd, k2


def kernel(x, edge_index, batch, W1, b1, W2, b2, Wl, bl):
    src_f, dst_f = _prep(edge_index)
    src2 = src_f.reshape(NS, EPT)
    dst2 = dst_f.reshape(NS, EPT)

    x4 = jnp.pad(x, ((0, N_PAD - N_NODES), (0, 0))).reshape(
        N_PAD // 4, 4 * F_IN
    )
    batch4 = jnp.pad(
        batch, (0, N_PAD - N_NODES), constant_values=N_GRAPHS
    ).reshape(N_PAD // 4, 4)

    onz = jnp.concatenate(
        [jnp.ones((CHUNK,), jnp.float32), jnp.zeros((CHUNK,), jnp.float32)]
    )
    zeros_rows = jnp.zeros((CHUNK, HALF), jnp.float32)
    w1bd, k2 = _packed_weights(W1, W2)
    b1p = jnp.tile(_split(b1.reshape(1, F_HID)), (1, 1, 4))
    b2p = jnp.tile(_split(b2.reshape(1, F_HID)), (1, 1, 4))

    zp = _tcz(x4, w1bd)
    deg_flat = _sc_degree(dst2, onz)
    degq = (
        deg_flat.reshape(NC, N_PAD // 4, 4)
        .transpose(1, 0, 2)
        .reshape(N_PAD // 4, 8)
    )

    dp, g1p = _tc1(degq, zp)

    acc1 = _sc_agg(g1p.reshape(2, N_PAD, HALF), src2, dst2, zeros_rows)
    g2p = _tc2(acc1.reshape(2, N_PAD // 4, 128), g1p, dp, b1p, k2)

    acc2 = _sc_agg(g2p.reshape(2, N_PAD, HALF), src2, dst2, zeros_rows)
    out = _tc3(
        acc2.reshape(2, N_PAD // 4, 128), g2p, dp, b2p, batch4, Wl,
        bl.reshape(1, N_CLS),
    )
    return out


# speedup vs baseline: 3.0478x; 1.0089x over previous
import functools

import jax
import jax.numpy as jnp
from jax import lax
from jax.experimental import pallas as pl
from jax.experimental.pallas import tpu as pltpu
from jax.experimental.pallas import tpu_sc as plsc

N_NODES = 10000
N_EDGES = 320000
F_IN = 128
F_HID = 64
HALF = F_HID // 2
N_CLS = 3
N_GRAPHS = 64

NC = 2
NS = 16
N_PAD = 10240
PAD_IDX = N_PAD - 1
RPS = N_PAD // NS
CHUNK = 128
EPT = N_EDGES // NS
CPT = 160
EPC = N_EDGES // NS // NC
DPC = 80
NBUF = 4
ROW_BLK = 1024

_mesh = functools.partial(
    plsc.VectorSubcoreMesh, core_axis_name="c", subcore_axis_name="s"
)
_sc_params = pltpu.CompilerParams(use_tc_tiling_on_sc=False)


def _fill_pad(ref, start, count):
    def body(k, carry):
        ref[pl.ds(start + k * 16, 16)] = jnp.full((16,), PAD_IDX, jnp.int32)
        return carry

    lax.fori_loop(0, count, body, 0)


def _sc_degree(dst2, onz):
    @functools.partial(
        pl.kernel,
        mesh=_mesh(),
        compiler_params=_sc_params,
        out_type=jax.ShapeDtypeStruct((NC * N_PAD,), jnp.float32),
        scratch_types=[
            pltpu.VMEM((DPC * CHUNK,), jnp.int32),
            pltpu.VMEM((CHUNK,), jnp.float32),
            pltpu.VMEM((CHUNK,), jnp.float32),
            pltpu.VMEM_SHARED((N_PAD,), jnp.float32),
            pltpu.SemaphoreType.DMA,
        ],
    )
    def k(dst_hbm, onz_hbm, out_hbm, idx_a, ones_v, stage_v, acc_sh, sem):
        c = lax.axis_index("c")
        s = lax.axis_index("s")
        pltpu.sync_copy(onz_hbm.at[pl.ds(0, CHUNK)], ones_v)
        pltpu.sync_copy(onz_hbm.at[pl.ds(CHUNK, CHUNK)], stage_v)

        def zbody(j, carry):
            pltpu.sync_copy(
                stage_v, acc_sh.at[pl.ds(s * RPS + j * CHUNK, CHUNK)]
            )
            return carry

        lax.fori_loop(0, RPS // CHUNK, zbody, 0)
        pltpu.sync_copy(
            dst_hbm.at[s].at[pl.ds(c * EPC, EPC)], idx_a.at[pl.ds(0, EPC)]
        )
        _fill_pad(idx_a, EPC, (DPC * CHUNK - EPC) // 16)
        plsc.subcore_barrier()

        def body(o, carry):
            for j in range(16):
                pltpu.async_copy(
                    ones_v,
                    acc_sh.at[idx_a.at[pl.ds((o * 16 + j) * CHUNK, CHUNK)]],
                    sem,
                    add=True,
                )
            for j in range(16):
                pltpu.make_async_copy(
                    ones_v, acc_sh.at[idx_a.at[pl.ds(0, CHUNK)]], sem
                ).wait()
            return carry

        lax.fori_loop(0, DPC // 16, body, 0)
        plsc.subcore_barrier()

        def obody(j, carry):
            off = s * RPS + j * CHUNK
            pltpu.sync_copy(acc_sh.at[pl.ds(off, CHUNK)], stage_v)
            pltpu.sync_copy(stage_v, out_hbm.at[pl.ds(c * N_PAD + off, CHUNK)])
            return carry

        lax.fori_loop(0, RPS // CHUNK, obody, 0)

    return k(dst2, onz)


def _sc_agg(g_sp, src2, dst2, zeros_rows):
    @functools.partial(
        pl.kernel,
        mesh=_mesh(),
        compiler_params=_sc_params,
        out_type=jax.ShapeDtypeStruct((NC, N_PAD, HALF), jnp.float32),
        scratch_types=[
            pltpu.VMEM((CPT * CHUNK,), jnp.int32),
            pltpu.VMEM((CPT * CHUNK,), jnp.int32),
            [pltpu.VMEM((CHUNK, HALF), jnp.float32)] * NBUF,
            [pltpu.SemaphoreType.DMA] * NBUF,
            [pltpu.SemaphoreType.DMA] * NBUF,
            pltpu.VMEM_SHARED((N_PAD, HALF), jnp.float32),
            pltpu.VMEM_SHARED((N_PAD, HALF), jnp.float32),
        ],
    )
    def k(g_hbm, src_hbm, dst_hbm, z_hbm, out_hbm,
          src_a, dst_a, rows, gsem, ssem, acc_sh, g_sh):
        c = lax.axis_index("c")
        s = lax.axis_index("s")

        nz = RPS // CHUNK
        sl = lambda j: pl.ds(s * RPS + j * CHUNK, CHUNK)
        pltpu.sync_copy(z_hbm, rows[0])
        pltpu.async_copy(src_hbm.at[s], src_a.at[pl.ds(0, EPT)], gsem[0])
        pltpu.async_copy(dst_hbm.at[s], dst_a.at[pl.ds(0, EPT)], gsem[0])
        for j in range(nz):
            pltpu.async_copy(rows[0], acc_sh.at[sl(j)], ssem[0])
        for j in range(nz):
            b = 1 + (j % 3)
            if j >= 3:
                pltpu.make_async_copy(
                    rows[b], g_sh.at[sl(j - 3)], ssem[b]
                ).wait()
            pltpu.async_copy(g_hbm.at[c].at[sl(j)], rows[b], gsem[b])
            pltpu.make_async_copy(
                g_hbm.at[c].at[sl(j)], rows[b], gsem[b]
            ).wait()
            pltpu.async_copy(rows[b], g_sh.at[sl(j)], ssem[b])
        for j in range(nz - 3, nz):
            b = 1 + (j % 3)
            pltpu.make_async_copy(rows[b], g_sh.at[sl(j)], ssem[b]).wait()
        for j in range(nz):
            pltpu.make_async_copy(rows[0], acc_sh.at[sl(j)], ssem[0]).wait()
        pltpu.make_async_copy(
            src_hbm.at[s], src_a.at[pl.ds(0, EPT)], gsem[0]
        ).wait()
        pltpu.make_async_copy(
            dst_hbm.at[s], dst_a.at[pl.ds(0, EPT)], gsem[0]
        ).wait()
        _fill_pad(src_a, EPT, (CPT * CHUNK - EPT) // 16)
        _fill_pad(dst_a, EPT, (CPT * CHUNK - EPT) // 16)
        plsc.subcore_barrier()

        for b in range(NBUF):
            pltpu.async_copy(
                g_sh.at[src_a.at[pl.ds(b * CHUNK, CHUNK)]], rows[b], gsem[b]
            )

        def outer(o, carry):
            for b in range(NBUF):
                i = o * NBUF + b
                pltpu.make_async_copy(
                    g_sh.at[src_a.at[pl.ds(0, CHUNK)]], rows[b], gsem[b]
                ).wait()
                pltpu.async_copy(
                    rows[b],
                    acc_sh.at[dst_a.at[pl.ds(i * CHUNK, CHUNK)]],
                    ssem[b],
                    add=True,
                )
                pltpu.make_async_copy(
                    rows[b],
                    acc_sh.at[dst_a.at[pl.ds(i * CHUNK, CHUNK)]],
                    ssem[b],
                ).wait()

                @pl.when(i + NBUF < CPT)
                def _():
                    pltpu.async_copy(
                        g_sh.at[src_a.at[pl.ds((i + NBUF) * CHUNK, CHUNK)]],
                        rows[b],
                        gsem[b],
                    )

            return carry

        lax.fori_loop(0, CPT // NBUF, outer, 0)
        plsc.subcore_barrier()

        def obody(j, carry):
            off = s * RPS + j * CHUNK
            pltpu.sync_copy(acc_sh.at[pl.ds(off, CHUNK)], rows[0])
            pltpu.sync_copy(rows[0], out_hbm.at[c].at[pl.ds(off, CHUNK)])
            return carry

        lax.fori_loop(0, RPS // CHUNK, obody, 0)

    return k(g_sp, src2, dst2, zeros_rows)


def _split(v):
    return jnp.stack([v[:, :HALF], v[:, HALF:]], axis=0)


def _pack(v):
    return v.reshape(2, v.shape[1] // 4, 128)


def _prep_body(ei_ref, s_ref, d_ref):
    s_ref[...] = ei_ref[0, :]
    d_ref[...] = ei_ref[1, :]


def _prep(edge_index):
    ob = pl.BlockSpec((N_EDGES,), lambda: (0,))
    return pl.pallas_call(
        _prep_body,
        in_specs=[pl.BlockSpec((2, N_EDGES), lambda: (0, 0))],
        out_specs=[ob, ob],
        out_shape=[
            jax.ShapeDtypeStruct((N_EDGES,), jnp.int32),
            jax.ShapeDtypeStruct((N_EDGES,), jnp.int32),
        ],
    )(edge_index)


def _tcz_body(x_ref, w_ref, z_ref):
    z_ref[...] = jnp.dot(
        x_ref[...], w_ref[...], preferred_element_type=jnp.float32
    )


def _tcz(x4, w1bd):
    grid = N_PAD // ROW_BLK
    r4 = ROW_BLK // 4
    return pl.pallas_call(
        _tcz_body,
        grid=(grid,),
        in_specs=[
            pl.BlockSpec((r4, 4 * F_IN), lambda i: (i, 0)),
            pl.BlockSpec((4 * F_IN, 256), lambda i: (0, 0)),
        ],
        out_specs=pl.BlockSpec((r4, 256), lambda i: (i, 0)),
        out_shape=jax.ShapeDtypeStruct((N_PAD // 4, 256), jnp.float32),
    )(x4, w1bd)


def _lane_groups():
    row = lax.broadcasted_iota(jnp.int32, (4, 128), 0)
    lane = lax.broadcasted_iota(jnp.int32, (4, 128), 1)
    return (lane // HALF == row).astype(jnp.float32)


def _tc1_body(degq_ref, zp_ref, dp_ref, g1_ref):
    degq = degq_ref[...]
    deg4 = degq[:, 0:4] + degq[:, 4:8] + 1.0
    dis4 = lax.rsqrt(deg4)
    dp = jnp.dot(dis4, _lane_groups(), preferred_element_type=jnp.float32)
    dp_ref[...] = dp
    zp = zp_ref[...]
    g1_ref[...] = jnp.stack([zp[:, :128] * dp, zp[:, 128:] * dp], axis=0)


def _tc1(degq, zp):
    grid = N_PAD // ROW_BLK
    r4 = ROW_BLK // 4
    return pl.pallas_call(
        _tc1_body,
        grid=(grid,),
        in_specs=[
            pl.BlockSpec((r4, 8), lambda i: (i, 0)),
            pl.BlockSpec((r4, 256), lambda i: (i, 0)),
        ],
        out_specs=[
            pl.BlockSpec((r4, 128), lambda i: (i, 0)),
            pl.BlockSpec((2, r4, 128), lambda i: (0, i, 0)),
        ],
        out_shape=[
            jax.ShapeDtypeStruct((N_PAD // 4, 128), jnp.float32),
            jax.ShapeDtypeStruct((2, N_PAD // 4, 128), jnp.float32),
        ],
    )(degq, zp)


def _tc2_body(acc_ref, g1_ref, dp_ref, b1_ref, k2_ref, g2_ref):
    dp = dp_ref[...]
    pre = (acc_ref[...] + g1_ref[...]) * dp[None] + b1_ref[...]
    pre = jnp.maximum(pre, 0.0)
    h_cat = jnp.concatenate([pre[0], pre[1]], axis=1)
    z2p = jnp.dot(h_cat, k2_ref[...], preferred_element_type=jnp.float32)
    g2_ref[...] = jnp.stack([z2p[:, :128] * dp, z2p[:, 128:] * dp], axis=0)


def _tc2(acc, g1, dp, b1p, k2):
    grid = N_PAD // ROW_BLK
    r4 = ROW_BLK // 4
    sb = pl.BlockSpec((2, r4, 128), lambda i: (0, i, 0))
    return pl.pallas_call(
        _tc2_body,
        grid=(grid,),
        in_specs=[
            sb,
            sb,
            pl.BlockSpec((r4, 128), lambda i: (i, 0)),
            pl.BlockSpec((2, 1, 128), lambda i: (0, 0, 0)),
            pl.BlockSpec((256, 256), lambda i: (0, 0)),
        ],
        out_specs=sb,
        out_shape=jax.ShapeDtypeStruct((2, N_PAD // 4, 128), jnp.float32),
    )(acc, g1, dp, b1p, k2)


def _tc3_body(
    acc_ref, g2_ref, dp_ref, b2_ref, batch4_ref, wl_ref, bl_ref,
    out_ref, pooled_ref, cnt_ref,
):
    i = pl.program_id(0)

    @pl.when(i == 0)
    def _():
        pooled_ref[...] = jnp.zeros_like(pooled_ref)
        cnt_ref[...] = jnp.zeros_like(cnt_ref)

    pre = (acc_ref[...] + g2_ref[...]) * dp_ref[...][None] + b2_ref[...]
    pre = jnp.maximum(pre, 0.0)
    ids4 = batch4_ref[...]
    giota = lax.broadcasted_iota(jnp.int32, (1, N_GRAPHS), 1)
    dn = (((0,), (0,)), ((), ()))
    r4 = pre.shape[1]
    oh_sum = jnp.zeros((r4, N_GRAPHS), jnp.float32)
    for k in range(4):
        oh_k = (ids4[:, k : k + 1] == giota).astype(jnp.float32)
        oh_sum = oh_sum + oh_k
        h_k = jnp.concatenate(
            [
                pre[0][:, k * HALF : (k + 1) * HALF],
                pre[1][:, k * HALF : (k + 1) * HALF],
            ],
            axis=1,
        )
        pooled_ref[...] += lax.dot_general(
            oh_k, h_k, dn, preferred_element_type=jnp.float32
        )
    cnt_ref[...] += lax.dot_general(
        oh_sum, jnp.ones((r4, 1), jnp.float32), dn,
        preferred_element_type=jnp.float32,
    )

    @pl.when(i == pl.num_programs(0) - 1)
    def _():
        mean = pooled_ref[...] / jnp.maximum(cnt_ref[...], 1.0)
        out_ref[...] = (
            jnp.dot(mean, wl_ref[...], preferred_element_type=jnp.float32)
            + bl_ref[...]
        )


def _tc3(acc, g2, dp, b2p, batch4, wl, bl):
    grid = N_PAD // ROW_BLK
    r4 = ROW_BLK // 4
    sb = pl.BlockSpec((2, r4, 128), lambda i: (0, i, 0))
    return pl.pallas_call(
        _tc3_body,
        grid=(grid,),
        in_specs=[
            sb,
            sb,
            pl.BlockSpec((r4, 128), lambda i: (i, 0)),
            pl.BlockSpec((2, 1, 128), lambda i: (0, 0, 0)),
            pl.BlockSpec((r4, 4), lambda i: (i, 0)),
            pl.BlockSpec((F_HID, N_CLS), lambda i: (0, 0)),
            pl.BlockSpec((1, N_CLS), lambda i: (0, 0)),
        ],
        out_specs=pl.BlockSpec((N_GRAPHS, N_CLS), lambda i: (0, 0)),
        out_shape=jax.ShapeDtypeStruct((N_GRAPHS, N_CLS), jnp.float32),
        scratch_shapes=[
            pltpu.VMEM((N_GRAPHS, F_HID), jnp.float32),
            pltpu.VMEM((N_GRAPHS, 1), jnp.float32),
        ],
    )(acc, g2, dp, b2p, batch4, wl, bl)


def _packed_weights(W1, W2):
    import numpy as np

    eye4 = jnp.eye(4, dtype=jnp.float32)
    perm = np.concatenate(
        [
            np.concatenate([np.arange(HALF) + F_HID * k for k in range(4)]),
            np.concatenate(
                [np.arange(HALF) + HALF + F_HID * k for k in range(4)]
            ),
        ]
    )
    w1bd = jnp.kron(eye4, W1)[:, perm]
    k2 = jnp.concatenate(
        [
            jnp.concatenate(
                [jnp.kron(eye4, W2[:HALF, :HALF]),
                 jnp.kron(eye4, W2[:HALF, HALF:])], axis=1
            ),
            jnp.concatenate(
                [jnp.kron(eye4, W2[HALF:, :HALF]),
                 jnp.kron(eye4, W2[HALF:, HALF:])], axis=1
            ),
        ],
        axis=0,
    )
    return w1bd, k2


def kernel(x, edge_index, batch, W1, b1, W2, b2, Wl, bl):
    src_f, dst_f = _prep(edge_index)
    src2 = src_f.reshape(NS, EPT)
    dst2 = dst_f.reshape(NS, EPT)

    x4 = jnp.pad(x, ((0, N_PAD - N_NODES), (0, 0))).reshape(
        N_PAD // 4, 4 * F_IN
    )
    batch4 = jnp.pad(
        batch, (0, N_PAD - N_NODES), constant_values=N_GRAPHS
    ).reshape(N_PAD // 4, 4)

    onz = jnp.concatenate(
        [jnp.ones((CHUNK,), jnp.float32), jnp.zeros((CHUNK,), jnp.float32)]
    )
    zeros_rows = jnp.zeros((CHUNK, HALF), jnp.float32)
    w1bd, k2 = _packed_weights(W1, W2)
    b1p = jnp.tile(_split(b1.reshape(1, F_HID)), (1, 1, 4))
    b2p = jnp.tile(_split(b2.reshape(1, F_HID)), (1, 1, 4))

    zp = _tcz(x4, w1bd)
    deg_flat = _sc_degree(dst2, onz)
    degq = (
        deg_flat.reshape(NC, N_PAD // 4, 4)
        .transpose(1, 0, 2)
        .reshape(N_PAD // 4, 8)
    )

    dp, g1p = _tc1(degq, zp)

    acc1 = _sc_agg(g1p.reshape(2, N_PAD, HALF), src2, dst2, zeros_rows)
    g2p = _tc2(acc1.reshape(2, N_PAD // 4, 128), g1p, dp, b1p, k2)

    acc2 = _sc_agg(g2p.reshape(2, N_PAD, HALF), src2, dst2, zeros_rows)
    out = _tc3(
        acc2.reshape(2, N_PAD // 4, 128), g2p, dp, b2p, batch4, Wl,
        bl.reshape(1, N_CLS),
    )
    return out


# speedup vs baseline: 3.0502x; 1.0008x over previous
import functools

import jax
import jax.numpy as jnp
from jax import lax
from jax.experimental import pallas as pl
from jax.experimental.pallas import tpu as pltpu
from jax.experimental.pallas import tpu_sc as plsc

N_NODES = 10000
N_EDGES = 320000
F_IN = 128
F_HID = 64
HALF = F_HID // 2
N_CLS = 3
N_GRAPHS = 64

NC = 2
NS = 16
N_PAD = 10240
PAD_IDX = N_PAD - 1
RPS = N_PAD // NS
CHUNK = 128
EPT = N_EDGES // NS
CPT = 160
EPC = N_EDGES // NS // NC
DPC = 80
NBUF = 4
ROW_BLK = 1024

_mesh = functools.partial(
    plsc.VectorSubcoreMesh, core_axis_name="c", subcore_axis_name="s"
)
_sc_params = pltpu.CompilerParams(use_tc_tiling_on_sc=False)


def _fill_pad(ref, start, count):
    def body(k, carry):
        ref[pl.ds(start + k * 16, 16)] = jnp.full((16,), PAD_IDX, jnp.int32)
        return carry

    lax.fori_loop(0, count, body, 0)


def _sc_degree(dst2, onz):
    @functools.partial(
        pl.kernel,
        mesh=_mesh(),
        compiler_params=_sc_params,
        out_type=jax.ShapeDtypeStruct((NC * N_PAD,), jnp.float32),
        scratch_types=[
            pltpu.VMEM((DPC * CHUNK,), jnp.int32),
            pltpu.VMEM((CHUNK,), jnp.float32),
            pltpu.VMEM((CHUNK,), jnp.float32),
            pltpu.VMEM_SHARED((N_PAD,), jnp.float32),
            pltpu.SemaphoreType.DMA,
        ],
    )
    def k(dst_hbm, onz_hbm, out_hbm, idx_a, ones_v, stage_v, acc_sh, sem):
        c = lax.axis_index("c")
        s = lax.axis_index("s")
        pltpu.sync_copy(onz_hbm.at[pl.ds(0, CHUNK)], ones_v)
        pltpu.sync_copy(onz_hbm.at[pl.ds(CHUNK, CHUNK)], stage_v)

        def zbody(j, carry):
            pltpu.sync_copy(
                stage_v, acc_sh.at[pl.ds(s * RPS + j * CHUNK, CHUNK)]
            )
            return carry

        lax.fori_loop(0, RPS // CHUNK, zbody, 0)
        pltpu.sync_copy(
            dst_hbm.at[s].at[pl.ds(c * EPC, EPC)], idx_a.at[pl.ds(0, EPC)]
        )
        _fill_pad(idx_a, EPC, (DPC * CHUNK - EPC) // 16)
        plsc.subcore_barrier()

        def body(o, carry):
            for j in range(16):
                pltpu.async_copy(
                    ones_v,
                    acc_sh.at[idx_a.at[pl.ds((o * 16 + j) * CHUNK, CHUNK)]],
                    sem,
                    add=True,
                )
            for j in range(16):
                pltpu.make_async_copy(
                    ones_v, acc_sh.at[idx_a.at[pl.ds(0, CHUNK)]], sem
                ).wait()
            return carry

        lax.fori_loop(0, DPC // 16, body, 0)
        plsc.subcore_barrier()

        def obody(j, carry):
            off = s * RPS + j * CHUNK
            pltpu.sync_copy(acc_sh.at[pl.ds(off, CHUNK)], stage_v)
            pltpu.sync_copy(stage_v, out_hbm.at[pl.ds(c * N_PAD + off, CHUNK)])
            return carry

        lax.fori_loop(0, RPS // CHUNK, obody, 0)

    return k(dst2, onz)


def _sc_agg(g_sp, src2, dst2, zeros_rows):
    @functools.partial(
        pl.kernel,
        mesh=_mesh(),
        compiler_params=_sc_params,
        out_type=jax.ShapeDtypeStruct((NC, N_PAD, HALF), jnp.float32),
        scratch_types=[
            pltpu.VMEM((CPT * CHUNK,), jnp.int32),
            pltpu.VMEM((CPT * CHUNK,), jnp.int32),
            [pltpu.VMEM((CHUNK, HALF), jnp.float32)] * NBUF,
            [pltpu.SemaphoreType.DMA] * NBUF,
            [pltpu.SemaphoreType.DMA] * NBUF,
            pltpu.VMEM_SHARED((N_PAD, HALF), jnp.float32),
            pltpu.VMEM_SHARED((N_PAD, HALF), jnp.float32),
        ],
    )
    def k(g_hbm, src_hbm, dst_hbm, z_hbm, out_hbm,
          src_a, dst_a, rows, gsem, ssem, acc_sh, g_sh):
        c = lax.axis_index("c")
        s = lax.axis_index("s")

        nz = RPS // CHUNK
        sl = lambda j: pl.ds(s * RPS + j * CHUNK, CHUNK)
        pltpu.sync_copy(z_hbm, rows[0])
        pltpu.async_copy(src_hbm.at[s], src_a.at[pl.ds(0, EPT)], gsem[0])
        pltpu.async_copy(dst_hbm.at[s], dst_a.at[pl.ds(0, EPT)], gsem[0])
        for j in range(nz):
            pltpu.async_copy(rows[0], acc_sh.at[sl(j)], ssem[0])
        for j in range(nz):
            b = 1 + (j % 3)
            if j >= 3:
                pltpu.make_async_copy(
                    rows[b], g_sh.at[sl(j - 3)], ssem[b]
                ).wait()
            pltpu.async_copy(g_hbm.at[c].at[sl(j)], rows[b], gsem[b])
            pltpu.make_async_copy(
                g_hbm.at[c].at[sl(j)], rows[b], gsem[b]
            ).wait()
            pltpu.async_copy(rows[b], g_sh.at[sl(j)], ssem[b])
        for j in range(nz - 3, nz):
            b = 1 + (j % 3)
            pltpu.make_async_copy(rows[b], g_sh.at[sl(j)], ssem[b]).wait()
        for j in range(nz):
            pltpu.make_async_copy(rows[0], acc_sh.at[sl(j)], ssem[0]).wait()
        pltpu.make_async_copy(
            src_hbm.at[s], src_a.at[pl.ds(0, EPT)], gsem[0]
        ).wait()
        pltpu.make_async_copy(
            dst_hbm.at[s], dst_a.at[pl.ds(0, EPT)], gsem[0]
        ).wait()
        _fill_pad(src_a, EPT, (CPT * CHUNK - EPT) // 16)
        _fill_pad(dst_a, EPT, (CPT * CHUNK - EPT) // 16)
        plsc.subcore_barrier()

        for b in range(NBUF):
            pltpu.async_copy(
                g_sh.at[src_a.at[pl.ds(b * CHUNK, CHUNK)]], rows[b], gsem[b]
            )

        def outer(o, carry):
            for b in range(NBUF):
                i = o * NBUF + b
                pltpu.make_async_copy(
                    g_sh.at[src_a.at[pl.ds(0, CHUNK)]], rows[b], gsem[b]
                ).wait()
                pltpu.async_copy(
                    rows[b],
                    acc_sh.at[dst_a.at[pl.ds(i * CHUNK, CHUNK)]],
                    ssem[b],
                    add=True,
                )
                pltpu.make_async_copy(
                    rows[b],
                    acc_sh.at[dst_a.at[pl.ds(i * CHUNK, CHUNK)]],
                    ssem[b],
                ).wait()

                @pl.when(i + NBUF < CPT)
                def _():
                    pltpu.async_copy(
                        g_sh.at[src_a.at[pl.ds((i + NBUF) * CHUNK, CHUNK)]],
                        rows[b],
                        gsem[b],
                    )

            return carry

        lax.fori_loop(0, CPT // NBUF, outer, 0)
        plsc.subcore_barrier()

        def obody(j, carry):
            off = s * RPS + j * CHUNK
            pltpu.sync_copy(acc_sh.at[pl.ds(off, CHUNK)], rows[0])
            pltpu.sync_copy(rows[0], out_hbm.at[c].at[pl.ds(off, CHUNK)])
            return carry

        lax.fori_loop(0, RPS // CHUNK, obody, 0)

    return k(g_sp, src2, dst2, zeros_rows)


def _split(v):
    return jnp.stack([v[:, :HALF], v[:, HALF:]], axis=0)


def _prep_body(ei_ref, s_ref, d_ref):
    s_ref[...] = ei_ref[0, :]
    d_ref[...] = ei_ref[1, :]


def _prep(edge_index):
    ob = pl.BlockSpec((N_EDGES,), lambda: (0,))
    return pl.pallas_call(
        _prep_body,
        in_specs=[pl.BlockSpec((2, N_EDGES), lambda: (0, 0))],
        out_specs=[ob, ob],
        out_shape=[
            jax.ShapeDtypeStruct((N_EDGES,), jnp.int32),
            jax.ShapeDtypeStruct((N_EDGES,), jnp.int32),
        ],
    )(edge_index)


def _tcz_body(x_ref, w_ref, z_ref):
    z_ref[...] = jnp.dot(
        x_ref[...], w_ref[...], preferred_element_type=jnp.float32
    )


def _tcz(x4, w1bd):
    grid = N_PAD // ROW_BLK
    r4 = ROW_BLK // 4
    return pl.pallas_call(
        _tcz_body,
        grid=(grid,),
        in_specs=[
            pl.BlockSpec((r4, 4 * F_IN), lambda i: (i, 0)),
            pl.BlockSpec((4 * F_IN, 256), lambda i: (0, 0)),
        ],
        out_specs=pl.BlockSpec((r4, 256), lambda i: (i, 0)),
        out_shape=jax.ShapeDtypeStruct((N_PAD // 4, 256), jnp.float32),
    )(x4, w1bd)


def _lane_groups():
    row = lax.broadcasted_iota(jnp.int32, (4, 128), 0)
    lane = lax.broadcasted_iota(jnp.int32, (4, 128), 1)
    return (lane // HALF == row).astype(jnp.float32)


def _tc1_body(degq_ref, zp_ref, dp_ref, g1_ref):
    degq = degq_ref[...]
    deg4 = degq[:, 0:4] + degq[:, 4:8] + 1.0
    dis4 = lax.rsqrt(deg4)
    dp = jnp.dot(dis4, _lane_groups(), preferred_element_type=jnp.float32)
    dp_ref[...] = dp
    zp = zp_ref[...]
    g1_ref[...] = jnp.stack([zp[:, :128] * dp, zp[:, 128:] * dp], axis=0)


def _tc1(degq, zp):
    grid = N_PAD // ROW_BLK
    r4 = ROW_BLK // 4
    return pl.pallas_call(
        _tc1_body,
        grid=(grid,),
        in_specs=[
            pl.BlockSpec((r4, 8), lambda i: (i, 0)),
            pl.BlockSpec((r4, 256), lambda i: (i, 0)),
        ],
        out_specs=[
            pl.BlockSpec((r4, 128), lambda i: (i, 0)),
            pl.BlockSpec((2, r4, 128), lambda i: (0, i, 0)),
        ],
        out_shape=[
            jax.ShapeDtypeStruct((N_PAD // 4, 128), jnp.float32),
            jax.ShapeDtypeStruct((2, N_PAD // 4, 128), jnp.float32),
        ],
    )(degq, zp)


def _tc2_body(acc_ref, g1_ref, dp_ref, b1_ref, k2_ref, g2_ref):
    dp = dp_ref[...]
    pre = (acc_ref[...] + g1_ref[...]) * dp[None] + b1_ref[...]
    pre = jnp.maximum(pre, 0.0)
    h_cat = jnp.concatenate([pre[0], pre[1]], axis=1)
    z2p = jnp.dot(h_cat, k2_ref[...], preferred_element_type=jnp.float32)
    g2_ref[...] = jnp.stack([z2p[:, :128] * dp, z2p[:, 128:] * dp], axis=0)


def _tc2(acc, g1, dp, b1p, k2):
    grid = N_PAD // ROW_BLK
    r4 = ROW_BLK // 4
    sb = pl.BlockSpec((2, r4, 128), lambda i: (0, i, 0))
    return pl.pallas_call(
        _tc2_body,
        grid=(grid,),
        in_specs=[
            sb,
            sb,
            pl.BlockSpec((r4, 128), lambda i: (i, 0)),
            pl.BlockSpec((2, 1, 128), lambda i: (0, 0, 0)),
            pl.BlockSpec((256, 256), lambda i: (0, 0)),
        ],
        out_specs=sb,
        out_shape=jax.ShapeDtypeStruct((2, N_PAD // 4, 128), jnp.float32),
    )(acc, g1, dp, b1p, k2)


def _tc3_body(
    acc_ref, g2_ref, dp_ref, b2_ref, batch4_ref, wl_ref, bl_ref,
    out_ref, pooled_ref, cnt_ref,
):
    i = pl.program_id(0)

    @pl.when(i == 0)
    def _():
        pooled_ref[...] = jnp.zeros_like(pooled_ref)
        cnt_ref[...] = jnp.zeros_like(cnt_ref)

    pre = (acc_ref[...] + g2_ref[...]) * dp_ref[...][None] + b2_ref[...]
    pre = jnp.maximum(pre, 0.0)
    ids4 = batch4_ref[...]
    giota = lax.broadcasted_iota(jnp.int32, (1, N_GRAPHS), 1)
    dn = (((0,), (0,)), ((), ()))
    r4 = pre.shape[1]
    oh_sum = jnp.zeros((r4, N_GRAPHS), jnp.float32)
    for k in range(4):
        oh_k = (ids4[:, k : k + 1] == giota).astype(jnp.float32)
        oh_sum = oh_sum + oh_k
        h_k = jnp.concatenate(
            [
                pre[0][:, k * HALF : (k + 1) * HALF],
                pre[1][:, k * HALF : (k + 1) * HALF],
            ],
            axis=1,
        )
        pooled_ref[...] += lax.dot_general(
            oh_k, h_k, dn, preferred_element_type=jnp.float32
        )
    cnt_ref[...] += lax.dot_general(
        oh_sum, jnp.ones((r4, 1), jnp.float32), dn,
        preferred_element_type=jnp.float32,
    )

    @pl.when(i == pl.num_programs(0) - 1)
    def _():
        mean = pooled_ref[...] / jnp.maximum(cnt_ref[...], 1.0)
        out_ref[...] = (
            jnp.dot(mean, wl_ref[...], preferred_element_type=jnp.float32)
            + bl_ref[...]
        )


def _tc3(acc, g2, dp, b2p, batch4, wl, bl):
    grid = N_PAD // ROW_BLK
    r4 = ROW_BLK // 4
    sb = pl.BlockSpec((2, r4, 128), lambda i: (0, i, 0))
    return pl.pallas_call(
        _tc3_body,
        grid=(grid,),
        in_specs=[
            sb,
            sb,
            pl.BlockSpec((r4, 128), lambda i: (i, 0)),
            pl.BlockSpec((2, 1, 128), lambda i: (0, 0, 0)),
            pl.BlockSpec((r4, 4), lambda i: (i, 0)),
            pl.BlockSpec((F_HID, N_CLS), lambda i: (0, 0)),
            pl.BlockSpec((1, N_CLS), lambda i: (0, 0)),
        ],
        out_specs=pl.BlockSpec((N_GRAPHS, N_CLS), lambda i: (0, 0)),
        out_shape=jax.ShapeDtypeStruct((N_GRAPHS, N_CLS), jnp.float32),
        scratch_shapes=[
            pltpu.VMEM((N_GRAPHS, F_HID), jnp.float32),
            pltpu.VMEM((N_GRAPHS, 1), jnp.float32),
        ],
    )(acc, g2, dp, b2p, batch4, wl, bl)


def _packed_weights(W1, W2):
    import numpy as np

    eye4 = jnp.eye(4, dtype=jnp.float32)
    perm = np.concatenate(
        [
            np.concatenate([np.arange(HALF) + F_HID * k for k in range(4)]),
            np.concatenate(
                [np.arange(HALF) + HALF + F_HID * k for k in range(4)]
            ),
        ]
    )
    w1bd = jnp.kron(eye4, W1)[:, perm]
    k2 = jnp.concatenate(
        [
            jnp.concatenate(
                [jnp.kron(eye4, W2[:HALF, :HALF]),
                 jnp.kron(eye4, W2[:HALF, HALF:])], axis=1
            ),
            jnp.concatenate(
                [jnp.kron(eye4, W2[HALF:, :HALF]),
                 jnp.kron(eye4, W2[HALF:, HALF:])], axis=1
            ),
        ],
        axis=0,
    )
    return w1bd, k2


def kernel(x, edge_index, batch, W1, b1, W2, b2, Wl, bl):
    src_f, dst_f = _prep(edge_index)
    src2 = src_f.reshape(NS, EPT)
    dst2 = dst_f.reshape(NS, EPT)

    x4 = jnp.pad(x, ((0, N_PAD - N_NODES), (0, 0))).reshape(
        N_PAD // 4, 4 * F_IN
    )
    batch4 = jnp.pad(
        batch, (0, N_PAD - N_NODES), constant_values=N_GRAPHS
    ).reshape(N_PAD // 4, 4)

    onz = jnp.concatenate(
        [jnp.ones((CHUNK,), jnp.float32), jnp.zeros((CHUNK,), jnp.float32)]
    )
    zeros_rows = jnp.zeros((CHUNK, HALF), jnp.float32)
    w1bd, k2 = _packed_weights(W1, W2)
    b1p = jnp.tile(_split(b1.reshape(1, F_HID)), (1, 1, 4))
    b2p = jnp.tile(_split(b2.reshape(1, F_HID)), (1, 1, 4))

    zp = _tcz(x4, w1bd)
    deg_flat = _sc_degree(dst2, onz)
    degq = (
        deg_flat.reshape(NC, N_PAD // 4, 4)
        .transpose(1, 0, 2)
        .reshape(N_PAD // 4, 8)
    )

    dp, g1p = _tc1(degq, zp)

    acc1 = _sc_agg(g1p.reshape(2, N_PAD, HALF), src2, dst2, zeros_rows)
    g2p = _tc2(acc1.reshape(2, N_PAD // 4, 128), g1p, dp, b1p, k2)

    acc2 = _sc_agg(g2p.reshape(2, N_PAD, HALF), src2, dst2, zeros_rows)
    out = _tc3(
        acc2.reshape(2, N_PAD // 4, 128), g2p, dp, b2p, batch4, Wl,
        bl.reshape(1, N_CLS),
    )
    return out
